# baseline (speedup 1.0000x reference)
import functools

import jax
import jax.numpy as jnp
from jax import lax
from jax.experimental import pallas as pl
from jax.experimental.pallas import tpu as pltpu
from jax.experimental.pallas import tpu_sc as plsc

N = 10000
E = 320000
IN = 128
D = 64
NW = 4
L = 8
TH = 4
WH = 4
H1 = 8
C1 = 32
OUT = 16

E_PAD = 327680
NROW = E_PAD // 128
NP16 = N + 16
HALF = N // 2
ACC_ROWS = HALF + 8

_PREC = jax.lax.Precision.HIGHEST
_MESH = plsc.VectorSubcoreMesh(core_axis_name="c", subcore_axis_name="s")


def _vsplat(vec, j):
    idx = jnp.full((16, 1), j, dtype=jnp.int32)
    return lax.gather(
        vec, idx,
        lax.GatherDimensionNumbers(offset_dims=(), collapsed_slice_dims=(0,),
                                   start_index_map=(0,)),
        (1,), mode=lax.GatherScatterMode.PROMISE_IN_BOUNDS)


def _proj_body(x_ref, w_ref, b_ref, o_ref):
    o_ref[...] = (
        jnp.dot(x_ref[...], w_ref[...], precision=_PREC,
                preferred_element_type=jnp.float32)
        + b_ref[...]
    )


def _project(x, W_in_p, b_in_p):
    BX = 1000
    return pl.pallas_call(
        _proj_body,
        grid=(N // BX,),
        in_specs=[
            pl.BlockSpec((BX, IN), lambda i: (i, 0)),
            pl.BlockSpec((IN, 128), lambda i: (0, 0)),
            pl.BlockSpec((1, 128), lambda i: (0, 0)),
        ],
        out_specs=pl.BlockSpec((BX, 128), lambda i: (i, 0)),
        out_shape=jax.ShapeDtypeStruct((N, 128), jnp.float32),
    )(x, W_in_p, b_in_p)


def _tf_body(tok_ref, deg_ref, wse_ref, wq_ref, wk_ref, wv_ref, wo_ref,
             g1_ref, be1_ref, g2_ref, be2_ref, wf1_ref, bf1_ref, wf2_ref,
             bf2_ref, wpool_ref, out_ref):
    R = tok_ref.shape[0]
    BN = R // (NW * L)
    B2 = BN * NW
    dh = D // TH

    def mm(a, b):
        return jnp.dot(a, b, precision=_PREC, preferred_element_type=jnp.float32)

    di = jax.lax.broadcasted_iota(jnp.int32, (D, TH), 0)
    hi = jax.lax.broadcasted_iota(jnp.int32, (D, TH), 1)
    hmask = jnp.where(di // dh == hi, 1.0, 0.0)
    r32 = jax.lax.broadcasted_iota(jnp.int32, (TH * L, TH * L), 0)
    c32 = jax.lax.broadcasted_iota(jnp.int32, (TH * L, TH * L), 1)
    perm_jh = jnp.where((r32 // TH == c32 % L) & (r32 % TH == c32 // L),
                        1.0, 0.0)

    se = deg_ref[...] * wse_ref[...]
    tok = tok_ref[...][:, :D]
    t0 = (tok.reshape(BN, NW * L, D) + se[:, None, :]).reshape(R, D)

    q = mm(t0, wq_ref[...])
    k = mm(t0, wk_ref[...])
    v = mm(t0, wv_ref[...])

    kr = k.reshape(B2, L, D)
    cols = []
    for j in range(L):
        kj = jnp.broadcast_to(kr[:, j][:, None, :], (B2, L, D)).reshape(R, D)
        cols.append(mm(q * kj, hmask))
    s32 = mm(jnp.concatenate(cols, axis=1), perm_jh)
    s32 = s32 * (1.0 / jnp.sqrt(jnp.float32(dh)))
    att_h = []
    for h in range(TH):
        sh = s32[:, h * L:(h + 1) * L]
        mx = jnp.max(sh, axis=-1, keepdims=True)
        exh = jnp.exp(sh - mx)
        att_h.append(exh / jnp.sum(exh, axis=-1, keepdims=True))

    vr = v.reshape(B2, L, D)
    acc = jnp.zeros((R, D), jnp.float32)
    for j in range(L):
        vj = jnp.broadcast_to(vr[:, j][:, None, :], (B2, L, D)).reshape(R, D)
        aj = jnp.concatenate(
            [jnp.broadcast_to(att_h[h][:, j:j + 1], (R, dh))
             for h in range(TH)], axis=1)
        acc = acc + aj * vj
    a = mm(acc, wo_ref[...])

    def ln(z, g, b):
        mu = jnp.mean(z, axis=-1, keepdims=True)
        var = jnp.mean(z * z, axis=-1, keepdims=True) - mu * mu
        return (z - mu) * jax.lax.rsqrt(var + 1e-5) * g + b

    t1 = ln(t0 + a, g1_ref[...], be1_ref[...])
    f = mm(jax.nn.relu(mm(t1, wf1_ref[...]) + bf1_ref[...]), wf2_ref[...]) \
        + bf2_ref[...]
    t2 = ln(t1 + f, g2_ref[...], be2_ref[...])

    t2r = t2.reshape(B2, L, D)
    wr = t2r[:, 0, :]
    for j in range(1, L):
        wr = wr + t2r[:, j, :]
    wr = wr * (1.0 / L)

    slg = mm(wr, wpool_ref[...])
    slgr = slg.reshape(BN, NW, WH)
    mx2 = slgr[:, 0, :]
    for w in range(1, NW):
        mx2 = jnp.maximum(mx2, slgr[:, w, :])
    exw = [jnp.exp(slgr[:, w, :] - mx2) for w in range(NW)]
    ssum = exw[0]
    for w in range(1, NW):
        ssum = ssum + exw[w]
    alpha = [jnp.mean(exw[w] / ssum, axis=-1, keepdims=True)
             for w in range(NW)]

    wrr = wr.reshape(BN, NW, D)
    pooled = alpha[0] * wrr[:, 0, :]
    for w in range(1, NW):
        pooled = pooled + alpha[w] * wrr[:, w, :]
    out_ref[...] = jax.nn.relu(pooled)


def _walk_transformer(tok_pad, deg, w_se, Wq, Wk, Wv, Wo, g1, be1, g2, be2,
                      Wf1, bf1, Wf2, bf2, w_pool):
    BN = 80
    full = lambda shape: pl.BlockSpec(shape, lambda i: tuple(0 for _ in shape))
    return pl.pallas_call(
        _tf_body,
        grid=(N // BN,),
        in_specs=[
            pl.BlockSpec((BN * NW * L, 128), lambda i: (i, 0)),
            pl.BlockSpec((BN, 1), lambda i: (i, 0)),
            full((1, D)),
            full((D, D)), full((D, D)), full((D, D)), full((D, D)),
            full((1, D)), full((1, D)), full((1, D)), full((1, D)),
            full((D, 2 * D)), full((1, 2 * D)), full((2 * D, D)), full((1, D)),
            full((D, WH)),
        ],
        out_specs=pl.BlockSpec((BN, D), lambda i: (i, 0)),
        out_shape=jax.ShapeDtypeStruct((N, D), jnp.float32),
    )(tok_pad, deg.reshape(N, 1), w_se, Wq, Wk, Wv, Wo,
      g1.reshape(1, D), be1.reshape(1, D), g2.reshape(1, D), be2.reshape(1, D),
      Wf1, bf1.reshape(1, 2 * D), Wf2, bf2.reshape(1, D), w_pool)


@functools.partial(
    pl.kernel,
    out_type=jax.ShapeDtypeStruct((E_PAD, 128), jnp.float32),
    mesh=_MESH,
    scratch_types=[
        pltpu.VMEM((8, 128), jnp.int32),
        pltpu.VMEM((128, 128), jnp.float32),
        pltpu.SemaphoreType.DMA,
    ],
)
def _sc_gather(table_hbm, idx_hbm, out_hbm, idx_v, rows_v, sem):
    core = lax.axis_index("c")
    sub = lax.axis_index("s")
    wid = core * 16 + sub

    def chunk(ci, _):
        base = pl.multiple_of(wid * 10240 + ci * 1024, 1024)
        pltpu.sync_copy(idx_hbm.at[pl.ds(pl.multiple_of(base // 128, 8), 8)],
                        idx_v)
        for j in range(8):
            pltpu.async_copy(table_hbm.at[idx_v.at[j]], rows_v, sem).wait()
            pltpu.sync_copy(rows_v, out_hbm.at[pl.ds(base + j * 128, 128)])
        return 0

    lax.fori_loop(0, 10, chunk, 0)


@functools.partial(
    pl.kernel,
    out_type=(jax.ShapeDtypeStruct((E_PAD // 8, 128), jnp.float32),
              jax.ShapeDtypeStruct((2, NP16, 128), jnp.float32)),
    mesh=_MESH,
    scratch_types=[
        pltpu.VMEM((8, 128), jnp.int32),
        pltpu.VMEM((8, 128), jnp.int32),
        pltpu.VMEM((64, 128), jnp.float32),
        pltpu.VMEM((64, 128), jnp.float32),
        pltpu.VMEM((128, 128), jnp.float32),
        pltpu.VMEM((16, 128), jnp.float32),
        pltpu.VMEM_SHARED((NP16, 128), jnp.float32),
        pltpu.SemaphoreType.DMA,
        pltpu.SemaphoreType.DMA,
    ],
)
def _sc_edge_stats(src_hbm, dst_hbm, als_hbm, ald_hbm, ex_hbm, s_hbm,
                   srcv, dstv, asv, adv, exw, exv, s_sh, sem1, sem2):
    core = lax.axis_index("c")
    sub = lax.axis_index("s")
    wid = core * 16 + sub

    def zrow(i, _):
        for j in range(8):
            exw[i, pl.ds(j * 16, 16)] = jnp.zeros((16,), jnp.float32)
        return 0
    lax.fori_loop(0, 128, zrow, 0)
    for k in range(5):
        pltpu.sync_copy(
            exw, s_sh.at[pl.ds(pl.multiple_of(sub * 624 + k * 128, 8), 128)])
    plsc.subcore_barrier()

    def chunk(ci, _):
        base = pl.multiple_of(wid * 10240 + ci * 1024, 1024)
        rb = pl.multiple_of(base // 128, 8)
        pltpu.sync_copy(src_hbm.at[pl.ds(rb, 8)], srcv)
        pltpu.sync_copy(dst_hbm.at[pl.ds(rb, 8)], dstv)

        def sblock(sb, _):
            for hb in range(2):
                c1 = pltpu.async_copy(
                    als_hbm.at[srcv.at[sb, pl.ds(hb * 64, 64)]], asv, sem1)
                c2 = pltpu.async_copy(
                    ald_hbm.at[dstv.at[sb, pl.ds(hb * 64, 64)]], adv, sem2)
                c1.wait()
                c2.wait()
                for r in range(8):
                    for j8 in range(8):
                        e64 = r * 8 + j8
                        e = hb * 64 + e64
                        z = asv[e64, pl.ds(0, 16)] + adv[e64, pl.ds(0, 16)]
                        zl = jnp.where(z > 0, z, z * 0.2)
                        ex16 = jnp.exp(zl)
                        exw[e, pl.ds(0, 16)] = ex16
                        exv[e // 8, pl.ds((e % 8) * 16, 16)] = ex16
            pltpu.sync_copy(exw, s_sh.at[dstv.at[sb]], add=True)
            pltpu.sync_copy(
                exv,
                ex_hbm.at[pl.ds(pl.multiple_of(base // 8 + sb * 16, 8), 16)])
            return 0
        lax.fori_loop(0, 8, sblock, 0)
        return 0

    lax.fori_loop(0, 10, chunk, 0)
    plsc.subcore_barrier()
    rb0 = pl.multiple_of(sub * 624, 8)
    pltpu.sync_copy(s_sh.at[pl.ds(rb0, 640)],
                    s_hbm.at[core, pl.ds(rb0, 640)])


@functools.partial(
    pl.kernel,
    out_type=jax.ShapeDtypeStruct((E_PAD // 8, 128), jnp.float32),
    mesh=_MESH,
    scratch_types=[
        pltpu.VMEM((8, 128), jnp.int32),
        pltpu.VMEM((128, 128), jnp.float32),
        pltpu.VMEM((128, 128), jnp.float32),
        pltpu.SemaphoreType.DMA,
    ],
)
def _sc_edge_w(dst_hbm, ex_hbm, r_hbm, w_hbm, dstv, exv, rrv, sem):
    core = lax.axis_index("c")
    sub = lax.axis_index("s")
    wid = core * 16 + sub

    def chunk(ci, _):
        base = pl.multiple_of(wid * 10240 + ci * 1024, 1024)
        rb = pl.multiple_of(base // 128, 8)
        pltpu.sync_copy(dst_hbm.at[pl.ds(rb, 8)], dstv)
        pltpu.sync_copy(ex_hbm.at[pl.ds(pl.multiple_of(base // 8, 8), 128)],
                        exv)

        def sblock(sb, _):
            pltpu.async_copy(r_hbm.at[dstv.at[sb]], rrv, sem).wait()
            for r in range(16):
                for j8 in range(8):
                    e = r * 8 + j8
                    fr = sb * 16 + r
                    exv[fr, pl.ds(j8 * 16, 16)] = (
                        exv[fr, pl.ds(j8 * 16, 16)] * rrv[e, pl.ds(0, 16)])
            return 0
        lax.fori_loop(0, 8, sblock, 0)
        pltpu.sync_copy(exv,
                        w_hbm.at[pl.ds(pl.multiple_of(base // 8, 8), 128)])
        return 0

    lax.fori_loop(0, 10, chunk, 0)


def _make_aggregate(wlanes):
    plan = []
    off = 0
    while off < 320:
        ln = min(128, 320 - off)
        plan.append((off, ln))
        off += ln

    @functools.partial(
        pl.kernel,
        out_type=jax.ShapeDtypeStruct((N, 128), jnp.float32),
        mesh=_MESH,
        scratch_types=[
            pltpu.VMEM((8, 128), jnp.int32),
            pltpu.VMEM((8, 128), jnp.int32),
            pltpu.VMEM((8, 128), jnp.int32),
            pltpu.VMEM((16, 128), jnp.float32),
            pltpu.VMEM((128, 128), jnp.float32),
            pltpu.VMEM_SHARED((ACC_ROWS, 128), jnp.float32),
            pltpu.SemaphoreType.DMA,
            pltpu.SemaphoreType.DMA,
        ],
    )
    def agg(src_hbm, dst_hbm, w_hbm, xp_hbm, out_hbm,
            srcv, dstv, ldv, wv_b, rows, acc, sem1, sem2):
        core = lax.axis_index("c")
        sub = lax.axis_index("s")
        base_n = core * HALF

        def zrow(i, _):
            for j in range(8):
                rows[i, pl.ds(j * 16, 16)] = jnp.zeros((16,), jnp.float32)
            return 0
        lax.fori_loop(0, 128, zrow, 0)
        for (o, ln) in plan:
            pltpu.sync_copy(
                rows.at[pl.ds(0, ln)],
                acc.at[pl.ds(pl.multiple_of(sub * 312 + o, 8), ln)])
        plsc.subcore_barrier()

        def group(ci, _):
            base = pl.multiple_of(sub * 20480 + ci * 1024, 1024)
            rb = pl.multiple_of(base // 128, 8)
            pltpu.sync_copy(src_hbm.at[pl.ds(rb, 8)], srcv)
            pltpu.sync_copy(dst_hbm.at[pl.ds(rb, 8)], dstv)
            for g in range(64):
                row = g // 8
                colo = (g % 8) * 16
                dv = dstv[row, pl.ds(colo, 16)]
                lv = dv - base_n
                inr = (lv >= 0) & (lv < HALF)
                ldv[row, pl.ds(colo, 16)] = jnp.where(inr, lv, HALF)

            def sblock(sb, _):
                c1 = pltpu.async_copy(xp_hbm.at[srcv.at[sb]], rows, sem1)
                pltpu.sync_copy(
                    w_hbm.at[pl.ds(
                        pl.multiple_of(base // 8 + sb * 16, 8), 16)], wv_b)
                c1.wait()
                for r in range(16):
                    for j8 in range(8):
                        e = r * 8 + j8
                        dvec = dstv[sb, pl.ds((e // 16) * 16, 16)]
                        d_s = dvec[e % 16]
                        inr = (d_s >= base_n) & (d_s < base_n + HALF)

                        @pl.when(inr)
                        def _():
                            wv = wv_b[r, pl.ds(j8 * 16, 16)]
                            splats = {}
                            for j in range(8):
                                ln_ = wlanes[j]
                                if ln_ not in splats:
                                    splats[ln_] = _vsplat(wv, ln_)
                                rows[e, pl.ds(j * 16, 16)] = (
                                    rows[e, pl.ds(j * 16, 16)] * splats[ln_])
                pltpu.sync_copy(rows, acc.at[ldv.at[sb]], add=True)
                return 0
            lax.fori_loop(0, 8, sblock, 0)
            return 0

        lax.fori_loop(0, 20, group, 0)
        plsc.subcore_barrier()
        for (o, ln) in plan:
            rloc = pl.multiple_of(sub * 312 + o, 8)
            pltpu.sync_copy(
                acc.at[pl.ds(rloc, ln)],
                out_hbm.at[pl.ds(pl.multiple_of(core * HALF + rloc, 8), ln)])

    return agg


_sc_agg1a = _make_aggregate([0, 0, 1, 1, 2, 2, 3, 3])
_sc_agg1b = _make_aggregate([4, 4, 5, 5, 6, 6, 7, 7])
_sc_agg2 = _make_aggregate([0] * 8)


def _prep1_body(gt_ref, w_ref, as_ref, ad_ref, xpa_ref, xpb_ref,
                als_ref, ald_ref):
    xp = jnp.dot(gt_ref[...], w_ref[...], precision=_PREC,
                 preferred_element_type=jnp.float32)
    xpa_ref[...] = xp[:, :128]
    xpb_ref[...] = xp[:, 128:]
    als_ref[...] = jnp.dot(xp, as_ref[...], precision=_PREC,
                           preferred_element_type=jnp.float32)
    ald_ref[...] = jnp.dot(xp, ad_ref[...], precision=_PREC,
                           preferred_element_type=jnp.float32)


def _prep1(gt, W1, As, Ad):
    BN = 1000
    F = H1 * C1
    return pl.pallas_call(
        _prep1_body,
        grid=(N // BN,),
        in_specs=[
            pl.BlockSpec((BN, D), lambda i: (i, 0)),
            pl.BlockSpec((D, F), lambda i: (0, 0)),
            pl.BlockSpec((F, 128), lambda i: (0, 0)),
            pl.BlockSpec((F, 128), lambda i: (0, 0)),
        ],
        out_specs=[
            pl.BlockSpec((BN, 128), lambda i: (i, 0)),
            pl.BlockSpec((BN, 128), lambda i: (i, 0)),
            pl.BlockSpec((BN, 128), lambda i: (i, 0)),
            pl.BlockSpec((BN, 128), lambda i: (i, 0)),
        ],
        out_shape=[
            jax.ShapeDtypeStruct((N, 128), jnp.float32),
            jax.ShapeDtypeStruct((N, 128), jnp.float32),
            jax.ShapeDtypeStruct((N, 128), jnp.float32),
            jax.ShapeDtypeStruct((N, 128), jnp.float32),
        ],
    )(gt, W1, As, Ad)


def _prep2_body(o1a_ref, o1b_ref, wa_ref, wb_ref, as_ref, ad_ref,
                xp_ref, als_ref, ald_ref):
    xp = (jnp.dot(jax.nn.relu(o1a_ref[...]), wa_ref[...], precision=_PREC,
                  preferred_element_type=jnp.float32)
          + jnp.dot(jax.nn.relu(o1b_ref[...]), wb_ref[...], precision=_PREC,
                    preferred_element_type=jnp.float32))
    xp_ref[...] = xp
    als_ref[...] = jnp.dot(xp, as_ref[...], precision=_PREC,
                           preferred_element_type=jnp.float32)
    ald_ref[...] = jnp.dot(xp, ad_ref[...], precision=_PREC,
                           preferred_element_type=jnp.float32)


def _prep2(o1a, o1b, W2pa, W2pb, As, Ad):
    BN = 1000
    return pl.pallas_call(
        _prep2_body,
        grid=(N // BN,),
        in_specs=[
            pl.BlockSpec((BN, 128), lambda i: (i, 0)),
            pl.BlockSpec((BN, 128), lambda i: (i, 0)),
            pl.BlockSpec((128, 128), lambda i: (0, 0)),
            pl.BlockSpec((128, 128), lambda i: (0, 0)),
            pl.BlockSpec((128, 128), lambda i: (0, 0)),
            pl.BlockSpec((128, 128), lambda i: (0, 0)),
        ],
        out_specs=[
            pl.BlockSpec((BN, 128), lambda i: (i, 0)),
            pl.BlockSpec((BN, 128), lambda i: (i, 0)),
            pl.BlockSpec((BN, 128), lambda i: (i, 0)),
        ],
        out_shape=[
            jax.ShapeDtypeStruct((N, 128), jnp.float32),
            jax.ShapeDtypeStruct((N, 128), jnp.float32),
            jax.ShapeDtypeStruct((N, 128), jnp.float32),
        ],
    )(o1a, o1b, W2pa, W2pb, As, Ad)


def _recip_body(s_ref, r_ref):
    r_ref[...] = 1.0 / (s_ref[0] + s_ref[1] + 1e-16)


def _recip(s):
    BR = 2504
    return pl.pallas_call(
        _recip_body,
        grid=(NP16 // BR,),
        in_specs=[pl.BlockSpec((2, BR, 128), lambda i: (0, i, 0))],
        out_specs=pl.BlockSpec((BR, 128), lambda i: (i, 0)),
        out_shape=jax.ShapeDtypeStruct((NP16, 128), jnp.float32),
    )(s)


def _softmax_body(x_ref, o_ref):
    z = x_ref[...][:, :OUT]
    m = jnp.max(z, axis=-1, keepdims=True)
    e = jnp.exp(z - m)
    o_ref[...] = e / jnp.sum(e, axis=-1, keepdims=True)


def _softmax16(x):
    return pl.pallas_call(
        _softmax_body,
        out_shape=jax.ShapeDtypeStruct((N, OUT), jnp.float32),
    )(x)


def kernel(x, edge_index, walks, deg, W_in, b_in, w_se, Wq, Wk, Wv, Wo, g1,
           be1, g2, be2, Wf1, bf1, Wf2, bf2, w_pool, W1, a_src1, a_dst1, W2,
           a_src2, a_dst2):
    src = edge_index[0]
    dst = edge_index[1]
    padE = E_PAD - E
    src2d = jnp.concatenate(
        [src, jnp.zeros((padE,), jnp.int32)]).reshape(NROW, 128)
    dst2d = jnp.concatenate(
        [dst, jnp.full((padE,), N, jnp.int32)]).reshape(NROW, 128)
    widx2d = jnp.concatenate(
        [walks.reshape(-1), jnp.zeros((padE,), jnp.int32)]).reshape(NROW, 128)

    F = H1 * C1
    W_in_p = jnp.concatenate([W_in, jnp.zeros((IN, 128 - D))], axis=1)
    b_in_p = jnp.concatenate([b_in, jnp.zeros((128 - D,))]).reshape(1, 128)
    W2p = jnp.concatenate([W2, jnp.zeros((F, 128 - OUT))], axis=1)
    W2pa = W2p[:128]
    W2pb = W2p[128:]
    As1 = jnp.zeros((F, 128), jnp.float32).at[
        jnp.arange(F), jnp.arange(F) // C1].set(a_src1.reshape(-1))
    Ad1 = jnp.zeros((F, 128), jnp.float32).at[
        jnp.arange(F), jnp.arange(F) // C1].set(a_dst1.reshape(-1))
    As2 = jnp.zeros((128, 128), jnp.float32).at[
        jnp.arange(OUT), 0].set(a_src2.reshape(-1))
    Ad2 = jnp.zeros((128, 128), jnp.float32).at[
        jnp.arange(OUT), 0].set(a_dst2.reshape(-1))
    zpad = jnp.zeros((16, 128), jnp.float32)

    xw = _project(x, W_in_p, b_in_p)
    tok_pad = _sc_gather(xw, widx2d)
    gt = _walk_transformer(tok_pad, deg, w_se, Wq, Wk, Wv, Wo, g1, be1, g2,
                           be2, Wf1, bf1, Wf2, bf2, w_pool)

    xp1a, xp1b, als1, ald1 = _prep1(gt, W1, As1, Ad1)
    ex1, s1 = _sc_edge_stats(src2d, dst2d, als1,
                             jnp.concatenate([ald1, zpad]))
    r1 = _recip(s1)
    w1 = _sc_edge_w(dst2d, ex1, r1)
    out1a = _sc_agg1a(src2d, dst2d, w1, xp1a)
    out1b = _sc_agg1b(src2d, dst2d, w1, xp1b)

    xp2, als2, ald2 = _prep2(out1a, out1b, W2pa, W2pb, As2, Ad2)
    ex2, s2 = _sc_edge_stats(src2d, dst2d, als2,
                             jnp.concatenate([ald2, zpad]))
    r2 = _recip(s2)
    w2 = _sc_edge_w(dst2d, ex2, r2)
    out2 = _sc_agg2(src2d, dst2d, w2, xp2)
    return _softmax16(out2)


# speedup vs baseline: 1.0910x; 1.0910x over previous
import functools

import jax
import jax.numpy as jnp
from jax import lax
from jax.experimental import pallas as pl
from jax.experimental.pallas import tpu as pltpu
from jax.experimental.pallas import tpu_sc as plsc

N = 10000
E = 320000
IN = 128
D = 64
NW = 4
L = 8
TH = 4
WH = 4
H1 = 8
C1 = 32
OUT = 16

E_PAD = 327680
NROW = E_PAD // 128
NP16 = N + 16
HALF = N // 2
ACC_ROWS = HALF + 8

_PREC = jax.lax.Precision.HIGHEST
_MESH = plsc.VectorSubcoreMesh(core_axis_name="c", subcore_axis_name="s")


def _vsplat(vec, j):
    idx = jnp.full((16, 1), j, dtype=jnp.int32)
    return lax.gather(
        vec, idx,
        lax.GatherDimensionNumbers(offset_dims=(), collapsed_slice_dims=(0,),
                                   start_index_map=(0,)),
        (1,), mode=lax.GatherScatterMode.PROMISE_IN_BOUNDS)


def _proj_body(x_ref, w_ref, b_ref, o_ref):
    o_ref[...] = (
        jnp.dot(x_ref[...], w_ref[...], precision=_PREC,
                preferred_element_type=jnp.float32)
        + b_ref[...]
    )


def _project(x, W_in_p, b_in_p):
    BX = 1000
    return pl.pallas_call(
        _proj_body,
        grid=(N // BX,),
        in_specs=[
            pl.BlockSpec((BX, IN), lambda i: (i, 0)),
            pl.BlockSpec((IN, 128), lambda i: (0, 0)),
            pl.BlockSpec((1, 128), lambda i: (0, 0)),
        ],
        out_specs=pl.BlockSpec((BX, 128), lambda i: (i, 0)),
        out_shape=jax.ShapeDtypeStruct((N, 128), jnp.float32),
    )(x, W_in_p, b_in_p)


def _tf_body(tok_ref, deg_ref, wse_ref, wq_ref, wk_ref, wv_ref, wo_ref,
             g1_ref, be1_ref, g2_ref, be2_ref, wf1_ref, bf1_ref, wf2_ref,
             bf2_ref, wpool_ref, out_ref):
    R = tok_ref.shape[0]
    BN = R // (NW * L)
    B2 = BN * NW
    dh = D // TH

    def mm(a, b):
        return jnp.dot(a, b, precision=_PREC, preferred_element_type=jnp.float32)

    di = jax.lax.broadcasted_iota(jnp.int32, (D, TH), 0)
    hi = jax.lax.broadcasted_iota(jnp.int32, (D, TH), 1)
    hmask = jnp.where(di // dh == hi, 1.0, 0.0)
    r32 = jax.lax.broadcasted_iota(jnp.int32, (TH * L, TH * L), 0)
    c32 = jax.lax.broadcasted_iota(jnp.int32, (TH * L, TH * L), 1)
    perm_jh = jnp.where((r32 // TH == c32 % L) & (r32 % TH == c32 // L),
                        1.0, 0.0)

    se = deg_ref[...] * wse_ref[...]
    tok = tok_ref[...][:, :D]
    t0 = (tok.reshape(BN, NW * L, D) + se[:, None, :]).reshape(R, D)

    q = mm(t0, wq_ref[...])
    k = mm(t0, wk_ref[...])
    v = mm(t0, wv_ref[...])

    kr = k.reshape(B2, L, D)
    cols = []
    for j in range(L):
        kj = jnp.broadcast_to(kr[:, j][:, None, :], (B2, L, D)).reshape(R, D)
        cols.append(mm(q * kj, hmask))
    s32 = mm(jnp.concatenate(cols, axis=1), perm_jh)
    s32 = s32 * (1.0 / jnp.sqrt(jnp.float32(dh)))
    att_h = []
    for h in range(TH):
        sh = s32[:, h * L:(h + 1) * L]
        mx = jnp.max(sh, axis=-1, keepdims=True)
        exh = jnp.exp(sh - mx)
        att_h.append(exh / jnp.sum(exh, axis=-1, keepdims=True))

    vr = v.reshape(B2, L, D)
    acc = jnp.zeros((R, D), jnp.float32)
    for j in range(L):
        vj = jnp.broadcast_to(vr[:, j][:, None, :], (B2, L, D)).reshape(R, D)
        aj = jnp.concatenate(
            [jnp.broadcast_to(att_h[h][:, j:j + 1], (R, dh))
             for h in range(TH)], axis=1)
        acc = acc + aj * vj
    a = mm(acc, wo_ref[...])

    def ln(z, g, b):
        mu = jnp.mean(z, axis=-1, keepdims=True)
        var = jnp.mean(z * z, axis=-1, keepdims=True) - mu * mu
        return (z - mu) * jax.lax.rsqrt(var + 1e-5) * g + b

    t1 = ln(t0 + a, g1_ref[...], be1_ref[...])
    f = mm(jax.nn.relu(mm(t1, wf1_ref[...]) + bf1_ref[...]), wf2_ref[...]) \
        + bf2_ref[...]
    t2 = ln(t1 + f, g2_ref[...], be2_ref[...])

    t2r = t2.reshape(B2, L, D)
    wr = t2r[:, 0, :]
    for j in range(1, L):
        wr = wr + t2r[:, j, :]
    wr = wr * (1.0 / L)

    slg = mm(wr, wpool_ref[...])
    slgr = slg.reshape(BN, NW, WH)
    mx2 = slgr[:, 0, :]
    for w in range(1, NW):
        mx2 = jnp.maximum(mx2, slgr[:, w, :])
    exw = [jnp.exp(slgr[:, w, :] - mx2) for w in range(NW)]
    ssum = exw[0]
    for w in range(1, NW):
        ssum = ssum + exw[w]
    alpha = [jnp.mean(exw[w] / ssum, axis=-1, keepdims=True)
             for w in range(NW)]

    wrr = wr.reshape(BN, NW, D)
    pooled = alpha[0] * wrr[:, 0, :]
    for w in range(1, NW):
        pooled = pooled + alpha[w] * wrr[:, w, :]
    out_ref[...] = jax.nn.relu(pooled)


def _walk_transformer(tok_pad, deg, w_se, Wq, Wk, Wv, Wo, g1, be1, g2, be2,
                      Wf1, bf1, Wf2, bf2, w_pool):
    BN = 80
    full = lambda shape: pl.BlockSpec(shape, lambda i: tuple(0 for _ in shape))
    return pl.pallas_call(
        _tf_body,
        grid=(N // BN,),
        in_specs=[
            pl.BlockSpec((BN * NW * L, 128), lambda i: (i, 0)),
            pl.BlockSpec((BN, 1), lambda i: (i, 0)),
            full((1, D)),
            full((D, D)), full((D, D)), full((D, D)), full((D, D)),
            full((1, D)), full((1, D)), full((1, D)), full((1, D)),
            full((D, 2 * D)), full((1, 2 * D)), full((2 * D, D)), full((1, D)),
            full((D, WH)),
        ],
        out_specs=pl.BlockSpec((BN, D), lambda i: (i, 0)),
        out_shape=jax.ShapeDtypeStruct((N, D), jnp.float32),
    )(tok_pad, deg.reshape(N, 1), w_se, Wq, Wk, Wv, Wo,
      g1.reshape(1, D), be1.reshape(1, D), g2.reshape(1, D), be2.reshape(1, D),
      Wf1, bf1.reshape(1, 2 * D), Wf2, bf2.reshape(1, D), w_pool)


@functools.partial(
    pl.kernel,
    out_type=jax.ShapeDtypeStruct((E_PAD, 128), jnp.float32),
    mesh=_MESH,
    scratch_types=[
        pltpu.VMEM((8, 128), jnp.int32),
        pltpu.VMEM((128, 128), jnp.float32),
        pltpu.SemaphoreType.DMA,
    ],
)
def _sc_gather(table_hbm, idx_hbm, out_hbm, idx_v, rows_v, sem):
    core = lax.axis_index("c")
    sub = lax.axis_index("s")
    wid = core * 16 + sub

    def chunk(ci, _):
        base = pl.multiple_of(wid * 10240 + ci * 1024, 1024)
        pltpu.sync_copy(idx_hbm.at[pl.ds(pl.multiple_of(base // 128, 8), 8)],
                        idx_v)
        for j in range(8):
            pltpu.async_copy(table_hbm.at[idx_v.at[j]], rows_v, sem).wait()
            pltpu.sync_copy(rows_v, out_hbm.at[pl.ds(base + j * 128, 128)])
        return 0

    lax.fori_loop(0, 10, chunk, 0)


@functools.partial(
    pl.kernel,
    out_type=(jax.ShapeDtypeStruct((E_PAD // 8, 128), jnp.float32),
              jax.ShapeDtypeStruct((2, NP16, 128), jnp.float32)),
    mesh=_MESH,
    scratch_types=[
        pltpu.VMEM((8, 128), jnp.int32),
        pltpu.VMEM((8, 128), jnp.int32),
        pltpu.VMEM((64, 128), jnp.float32),
        pltpu.VMEM((64, 128), jnp.float32),
        pltpu.VMEM((128, 128), jnp.float32),
        pltpu.VMEM((16, 128), jnp.float32),
        pltpu.VMEM_SHARED((NP16, 128), jnp.float32),
        pltpu.SemaphoreType.DMA,
        pltpu.SemaphoreType.DMA,
    ],
)
def _sc_edge_stats(src_hbm, dst_hbm, als_hbm, ald_hbm, ex_hbm, s_hbm,
                   srcv, dstv, asv, adv, exw, exv, s_sh, sem1, sem2):
    core = lax.axis_index("c")
    sub = lax.axis_index("s")
    wid = core * 16 + sub

    def zrow(i, _):
        for j in range(8):
            exw[i, pl.ds(j * 16, 16)] = jnp.zeros((16,), jnp.float32)
        return 0
    lax.fori_loop(0, 128, zrow, 0)
    for k in range(5):
        pltpu.sync_copy(
            exw, s_sh.at[pl.ds(pl.multiple_of(sub * 624 + k * 128, 8), 128)])
    plsc.subcore_barrier()

    def chunk(ci, _):
        base = pl.multiple_of(wid * 10240 + ci * 1024, 1024)
        rb = pl.multiple_of(base // 128, 8)
        pltpu.sync_copy(src_hbm.at[pl.ds(rb, 8)], srcv)
        pltpu.sync_copy(dst_hbm.at[pl.ds(rb, 8)], dstv)

        def sblock(sb, _):
            for hb in range(2):
                c1 = pltpu.async_copy(
                    als_hbm.at[srcv.at[sb, pl.ds(hb * 64, 64)]], asv, sem1)
                c2 = pltpu.async_copy(
                    ald_hbm.at[dstv.at[sb, pl.ds(hb * 64, 64)]], adv, sem2)
                c1.wait()
                c2.wait()
                for r in range(8):
                    for j8 in range(8):
                        e64 = r * 8 + j8
                        e = hb * 64 + e64
                        z = asv[e64, pl.ds(0, 16)] + adv[e64, pl.ds(0, 16)]
                        zl = jnp.where(z > 0, z, z * 0.2)
                        ex16 = jnp.exp(zl)
                        exw[e, pl.ds(0, 16)] = ex16
                        exv[e // 8, pl.ds((e % 8) * 16, 16)] = ex16
            pltpu.sync_copy(exw, s_sh.at[dstv.at[sb]], add=True)
            pltpu.sync_copy(
                exv,
                ex_hbm.at[pl.ds(pl.multiple_of(base // 8 + sb * 16, 8), 16)])
            return 0
        lax.fori_loop(0, 8, sblock, 0)
        return 0

    lax.fori_loop(0, 10, chunk, 0)
    plsc.subcore_barrier()
    rb0 = pl.multiple_of(sub * 624, 8)
    pltpu.sync_copy(s_sh.at[pl.ds(rb0, 640)],
                    s_hbm.at[core, pl.ds(rb0, 640)])


@functools.partial(
    pl.kernel,
    out_type=jax.ShapeDtypeStruct((E_PAD // 8, 128), jnp.float32),
    mesh=_MESH,
    scratch_types=[
        pltpu.VMEM((8, 128), jnp.int32),
        pltpu.VMEM((128, 128), jnp.float32),
        pltpu.VMEM((128, 128), jnp.float32),
        pltpu.SemaphoreType.DMA,
    ],
)
def _sc_edge_w(dst_hbm, ex_hbm, r_hbm, w_hbm, dstv, exv, rrv, sem):
    core = lax.axis_index("c")
    sub = lax.axis_index("s")
    wid = core * 16 + sub

    def chunk(ci, _):
        base = pl.multiple_of(wid * 10240 + ci * 1024, 1024)
        rb = pl.multiple_of(base // 128, 8)
        pltpu.sync_copy(dst_hbm.at[pl.ds(rb, 8)], dstv)
        pltpu.sync_copy(ex_hbm.at[pl.ds(pl.multiple_of(base // 8, 8), 128)],
                        exv)

        def sblock(sb, _):
            pltpu.async_copy(r_hbm.at[dstv.at[sb]], rrv, sem).wait()
            for r in range(16):
                for j8 in range(8):
                    e = r * 8 + j8
                    fr = sb * 16 + r
                    exv[fr, pl.ds(j8 * 16, 16)] = (
                        exv[fr, pl.ds(j8 * 16, 16)] * rrv[e, pl.ds(0, 16)])
            return 0
        lax.fori_loop(0, 8, sblock, 0)
        pltpu.sync_copy(exv,
                        w_hbm.at[pl.ds(pl.multiple_of(base // 8, 8), 128)])
        return 0

    lax.fori_loop(0, 10, chunk, 0)


def _make_aggregate(wlanes):
    plan = []
    off = 0
    while off < 320:
        ln = min(128, 320 - off)
        plan.append((off, ln))
        off += ln

    @functools.partial(
        pl.kernel,
        out_type=jax.ShapeDtypeStruct((N, 128), jnp.float32),
        mesh=_MESH,
        scratch_types=[
            pltpu.VMEM((8, 128), jnp.int32),
            pltpu.VMEM((8, 128), jnp.int32),
            pltpu.VMEM((8, 128), jnp.int32),
            pltpu.VMEM((128, 128), jnp.float32),
            pltpu.VMEM((128, 128), jnp.float32),
            pltpu.VMEM((128, 128), jnp.float32),
            pltpu.VMEM((128, 128), jnp.float32),
            pltpu.VMEM((128, 128), jnp.float32),
            pltpu.VMEM_SHARED((ACC_ROWS, 128), jnp.float32),
            pltpu.SemaphoreType.DMA,
            pltpu.SemaphoreType.DMA,
        ],
    )
    def agg(src_hbm, dst_hbm, w_hbm, xp_hbm, out_hbm,
            srcv, dstv, ldv, wv_g, rows0, rows1, rows2, rows3, acc,
            sem1, sem2):
        core = lax.axis_index("c")
        sub = lax.axis_index("s")
        base_n = core * HALF
        rowbufs = [rows0, rows1, rows2, rows3]

        def zrow(i, _):
            for j in range(8):
                rows0[i, pl.ds(j * 16, 16)] = jnp.zeros((16,), jnp.float32)
            return 0
        lax.fori_loop(0, 128, zrow, 0)
        for (o, ln) in plan:
            pltpu.sync_copy(
                rows0.at[pl.ds(0, ln)],
                acc.at[pl.ds(pl.multiple_of(sub * 312 + o, 8), ln)])
        plsc.subcore_barrier()

        def group(ci, _):
            base = pl.multiple_of(sub * 20480 + ci * 1024, 1024)
            rb = pl.multiple_of(base // 128, 8)
            pltpu.sync_copy(src_hbm.at[pl.ds(rb, 8)], srcv)
            pltpu.sync_copy(dst_hbm.at[pl.ds(rb, 8)], dstv)
            pltpu.sync_copy(
                w_hbm.at[pl.ds(pl.multiple_of(base // 8, 8), 128)], wv_g)
            for g in range(64):
                row = g // 8
                colo = (g % 8) * 16
                dv = dstv[row, pl.ds(colo, 16)]
                lv = dv - base_n
                inr = (lv >= 0) & (lv < HALF)
                ldv[row, pl.ds(colo, 16)] = jnp.where(inr, lv, HALF)
            for q in range(2):
                cps = [pltpu.async_copy(xp_hbm.at[srcv.at[q * 4 + b]],
                                        rowbufs[b], sem1)
                       for b in range(4)]
                for b in range(4):
                    sbid = q * 4 + b
                    rows = rowbufs[b]
                    cps[b].wait()

                    def rbody(r, _, sbid=sbid, rows=rows):
                        for j8 in range(8):
                            e = r * 8 + j8
                            wv = wv_g[sbid * 16 + r, pl.ds(j8 * 16, 16)]
                            splats = {}
                            for j in range(8):
                                ln_ = wlanes[j]
                                if ln_ not in splats:
                                    splats[ln_] = _vsplat(wv, ln_)
                                rows[e, pl.ds(j * 16, 16)] = (
                                    rows[e, pl.ds(j * 16, 16)] * splats[ln_])
                        return 0
                    lax.fori_loop(0, 16, rbody, 0)
                    pltpu.sync_copy(rows, acc.at[ldv.at[sbid]], add=True)
            return 0

        lax.fori_loop(0, 20, group, 0)
        plsc.subcore_barrier()
        for (o, ln) in plan:
            rloc = pl.multiple_of(sub * 312 + o, 8)
            pltpu.sync_copy(
                acc.at[pl.ds(rloc, ln)],
                out_hbm.at[pl.ds(pl.multiple_of(core * HALF + rloc, 8), ln)])

    return agg


_sc_agg1a = _make_aggregate([0, 0, 1, 1, 2, 2, 3, 3])
_sc_agg1b = _make_aggregate([4, 4, 5, 5, 6, 6, 7, 7])
_sc_agg2 = _make_aggregate([0] * 8)


def _prep1_body(gt_ref, w_ref, as_ref, ad_ref, xpa_ref, xpb_ref,
                als_ref, ald_ref):
    xp = jnp.dot(gt_ref[...], w_ref[...], precision=_PREC,
                 preferred_element_type=jnp.float32)
    xpa_ref[...] = xp[:, :128]
    xpb_ref[...] = xp[:, 128:]
    als_ref[...] = jnp.dot(xp, as_ref[...], precision=_PREC,
                           preferred_element_type=jnp.float32)
    ald_ref[...] = jnp.dot(xp, ad_ref[...], precision=_PREC,
                           preferred_element_type=jnp.float32)


def _prep1(gt, W1, As, Ad):
    BN = 1000
    F = H1 * C1
    return pl.pallas_call(
        _prep1_body,
        grid=(N // BN,),
        in_specs=[
            pl.BlockSpec((BN, D), lambda i: (i, 0)),
            pl.BlockSpec((D, F), lambda i: (0, 0)),
            pl.BlockSpec((F, 128), lambda i: (0, 0)),
            pl.BlockSpec((F, 128), lambda i: (0, 0)),
        ],
        out_specs=[
            pl.BlockSpec((BN, 128), lambda i: (i, 0)),
            pl.BlockSpec((BN, 128), lambda i: (i, 0)),
            pl.BlockSpec((BN, 128), lambda i: (i, 0)),
            pl.BlockSpec((BN, 128), lambda i: (i, 0)),
        ],
        out_shape=[
            jax.ShapeDtypeStruct((N, 128), jnp.float32),
            jax.ShapeDtypeStruct((N, 128), jnp.float32),
            jax.ShapeDtypeStruct((N, 128), jnp.float32),
            jax.ShapeDtypeStruct((N, 128), jnp.float32),
        ],
    )(gt, W1, As, Ad)


def _prep2_body(o1a_ref, o1b_ref, wa_ref, wb_ref, as_ref, ad_ref,
                xp_ref, als_ref, ald_ref):
    xp = (jnp.dot(jax.nn.relu(o1a_ref[...]), wa_ref[...], precision=_PREC,
                  preferred_element_type=jnp.float32)
          + jnp.dot(jax.nn.relu(o1b_ref[...]), wb_ref[...], precision=_PREC,
                    preferred_element_type=jnp.float32))
    xp_ref[...] = xp
    als_ref[...] = jnp.dot(xp, as_ref[...], precision=_PREC,
                           preferred_element_type=jnp.float32)
    ald_ref[...] = jnp.dot(xp, ad_ref[...], precision=_PREC,
                           preferred_element_type=jnp.float32)


def _prep2(o1a, o1b, W2pa, W2pb, As, Ad):
    BN = 1000
    return pl.pallas_call(
        _prep2_body,
        grid=(N // BN,),
        in_specs=[
            pl.BlockSpec((BN, 128), lambda i: (i, 0)),
            pl.BlockSpec((BN, 128), lambda i: (i, 0)),
            pl.BlockSpec((128, 128), lambda i: (0, 0)),
            pl.BlockSpec((128, 128), lambda i: (0, 0)),
            pl.BlockSpec((128, 128), lambda i: (0, 0)),
            pl.BlockSpec((128, 128), lambda i: (0, 0)),
        ],
        out_specs=[
            pl.BlockSpec((BN, 128), lambda i: (i, 0)),
            pl.BlockSpec((BN, 128), lambda i: (i, 0)),
            pl.BlockSpec((BN, 128), lambda i: (i, 0)),
        ],
        out_shape=[
            jax.ShapeDtypeStruct((N, 128), jnp.float32),
            jax.ShapeDtypeStruct((N, 128), jnp.float32),
            jax.ShapeDtypeStruct((N, 128), jnp.float32),
        ],
    )(o1a, o1b, W2pa, W2pb, As, Ad)


def _recip_body(s_ref, r_ref):
    r_ref[...] = 1.0 / (s_ref[0] + s_ref[1] + 1e-16)


def _recip(s):
    BR = 2504
    return pl.pallas_call(
        _recip_body,
        grid=(NP16 // BR,),
        in_specs=[pl.BlockSpec((2, BR, 128), lambda i: (0, i, 0))],
        out_specs=pl.BlockSpec((BR, 128), lambda i: (i, 0)),
        out_shape=jax.ShapeDtypeStruct((NP16, 128), jnp.float32),
    )(s)


def _softmax_body(x_ref, o_ref):
    z = x_ref[...][:, :OUT]
    m = jnp.max(z, axis=-1, keepdims=True)
    e = jnp.exp(z - m)
    o_ref[...] = e / jnp.sum(e, axis=-1, keepdims=True)


def _softmax16(x):
    return pl.pallas_call(
        _softmax_body,
        out_shape=jax.ShapeDtypeStruct((N, OUT), jnp.float32),
    )(x)


def kernel(x, edge_index, walks, deg, W_in, b_in, w_se, Wq, Wk, Wv, Wo, g1,
           be1, g2, be2, Wf1, bf1, Wf2, bf2, w_pool, W1, a_src1, a_dst1, W2,
           a_src2, a_dst2):
    src = edge_index[0]
    dst = edge_index[1]
    padE = E_PAD - E
    src2d = jnp.concatenate(
        [src, jnp.zeros((padE,), jnp.int32)]).reshape(NROW, 128)
    dst2d = jnp.concatenate(
        [dst, jnp.full((padE,), N, jnp.int32)]).reshape(NROW, 128)
    widx2d = jnp.concatenate(
        [walks.reshape(-1), jnp.zeros((padE,), jnp.int32)]).reshape(NROW, 128)

    F = H1 * C1
    W_in_p = jnp.concatenate([W_in, jnp.zeros((IN, 128 - D))], axis=1)
    b_in_p = jnp.concatenate([b_in, jnp.zeros((128 - D,))]).reshape(1, 128)
    W2p = jnp.concatenate([W2, jnp.zeros((F, 128 - OUT))], axis=1)
    W2pa = W2p[:128]
    W2pb = W2p[128:]
    As1 = jnp.zeros((F, 128), jnp.float32).at[
        jnp.arange(F), jnp.arange(F) // C1].set(a_src1.reshape(-1))
    Ad1 = jnp.zeros((F, 128), jnp.float32).at[
        jnp.arange(F), jnp.arange(F) // C1].set(a_dst1.reshape(-1))
    As2 = jnp.zeros((128, 128), jnp.float32).at[
        jnp.arange(OUT), 0].set(a_src2.reshape(-1))
    Ad2 = jnp.zeros((128, 128), jnp.float32).at[
        jnp.arange(OUT), 0].set(a_dst2.reshape(-1))
    zpad = jnp.zeros((16, 128), jnp.float32)

    xw = _project(x, W_in_p, b_in_p)
    tok_pad = _sc_gather(xw, widx2d)
    gt = _walk_transformer(tok_pad, deg, w_se, Wq, Wk, Wv, Wo, g1, be1, g2,
                           be2, Wf1, bf1, Wf2, bf2, w_pool)

    xp1a, xp1b, als1, ald1 = _prep1(gt, W1, As1, Ad1)
    ex1, s1 = _sc_edge_stats(src2d, dst2d, als1,
                             jnp.concatenate([ald1, zpad]))
    r1 = _recip(s1)
    w1 = _sc_edge_w(dst2d, ex1, r1)
    out1a = _sc_agg1a(src2d, dst2d, w1, xp1a)
    out1b = _sc_agg1b(src2d, dst2d, w1, xp1b)

    xp2, als2, ald2 = _prep2(out1a, out1b, W2pa, W2pb, As2, Ad2)
    ex2, s2 = _sc_edge_stats(src2d, dst2d, als2,
                             jnp.concatenate([ald2, zpad]))
    r2 = _recip(s2)
    w2 = _sc_edge_w(dst2d, ex2, r2)
    out2 = _sc_agg2(src2d, dst2d, w2, xp2)
    return _softmax16(out2)


# speedup vs baseline: 1.1848x; 1.0860x over previous
import functools

import jax
import jax.numpy as jnp
from jax import lax
from jax.experimental import pallas as pl
from jax.experimental.pallas import tpu as pltpu
from jax.experimental.pallas import tpu_sc as plsc

N = 10000
E = 320000
IN = 128
D = 64
NW = 4
L = 8
TH = 4
WH = 4
H1 = 8
C1 = 32
OUT = 16

E_PAD = 327680
NROW = E_PAD // 128
NP16 = N + 16
HALF = N // 2
ACC_ROWS = HALF + 8

_PREC = jax.lax.Precision.HIGHEST
_MESH = plsc.VectorSubcoreMesh(core_axis_name="c", subcore_axis_name="s")


def _vsplat(vec, j):
    idx = jnp.full((16, 1), j, dtype=jnp.int32)
    return lax.gather(
        vec, idx,
        lax.GatherDimensionNumbers(offset_dims=(), collapsed_slice_dims=(0,),
                                   start_index_map=(0,)),
        (1,), mode=lax.GatherScatterMode.PROMISE_IN_BOUNDS)


def _proj_body(x_ref, w_ref, b_ref, o_ref):
    o_ref[...] = (
        jnp.dot(x_ref[...], w_ref[...], precision=_PREC,
                preferred_element_type=jnp.float32)
        + b_ref[...]
    )


def _project(x, W_in_p, b_in_p):
    BX = 1000
    return pl.pallas_call(
        _proj_body,
        grid=(N // BX,),
        in_specs=[
            pl.BlockSpec((BX, IN), lambda i: (i, 0)),
            pl.BlockSpec((IN, 128), lambda i: (0, 0)),
            pl.BlockSpec((1, 128), lambda i: (0, 0)),
        ],
        out_specs=pl.BlockSpec((BX, 128), lambda i: (i, 0)),
        out_shape=jax.ShapeDtypeStruct((N, 128), jnp.float32),
    )(x, W_in_p, b_in_p)


def _tf_body(tok_ref, deg_ref, wse_ref, wq_ref, wk_ref, wv_ref, wo_ref,
             g1_ref, be1_ref, g2_ref, be2_ref, wf1_ref, bf1_ref, wf2_ref,
             bf2_ref, wpool_ref, out_ref):
    R = tok_ref.shape[0]
    BN = R // (NW * L)
    B2 = BN * NW
    dh = D // TH

    def mm(a, b):
        return jnp.dot(a, b, precision=_PREC, preferred_element_type=jnp.float32)

    di = jax.lax.broadcasted_iota(jnp.int32, (D, TH), 0)
    hi = jax.lax.broadcasted_iota(jnp.int32, (D, TH), 1)
    hmask = jnp.where(di // dh == hi, 1.0, 0.0)
    r32 = jax.lax.broadcasted_iota(jnp.int32, (TH * L, TH * L), 0)
    c32 = jax.lax.broadcasted_iota(jnp.int32, (TH * L, TH * L), 1)
    perm_jh = jnp.where((r32 // TH == c32 % L) & (r32 % TH == c32 // L),
                        1.0, 0.0)

    se = deg_ref[...] * wse_ref[...]
    tok = tok_ref[...][:, :D]
    t0 = (tok.reshape(BN, NW * L, D) + se[:, None, :]).reshape(R, D)

    q = mm(t0, wq_ref[...])
    k = mm(t0, wk_ref[...])
    v = mm(t0, wv_ref[...])

    kr = k.reshape(B2, L, D)
    cols = []
    for j in range(L):
        kj = jnp.broadcast_to(kr[:, j][:, None, :], (B2, L, D)).reshape(R, D)
        cols.append(mm(q * kj, hmask))
    s32 = mm(jnp.concatenate(cols, axis=1), perm_jh)
    s32 = s32 * (1.0 / jnp.sqrt(jnp.float32(dh)))
    att_h = []
    for h in range(TH):
        sh = s32[:, h * L:(h + 1) * L]
        mx = jnp.max(sh, axis=-1, keepdims=True)
        exh = jnp.exp(sh - mx)
        att_h.append(exh / jnp.sum(exh, axis=-1, keepdims=True))

    vr = v.reshape(B2, L, D)
    acc = jnp.zeros((R, D), jnp.float32)
    for j in range(L):
        vj = jnp.broadcast_to(vr[:, j][:, None, :], (B2, L, D)).reshape(R, D)
        aj = jnp.concatenate(
            [jnp.broadcast_to(att_h[h][:, j:j + 1], (R, dh))
             for h in range(TH)], axis=1)
        acc = acc + aj * vj
    a = mm(acc, wo_ref[...])

    def ln(z, g, b):
        mu = jnp.mean(z, axis=-1, keepdims=True)
        var = jnp.mean(z * z, axis=-1, keepdims=True) - mu * mu
        return (z - mu) * jax.lax.rsqrt(var + 1e-5) * g + b

    t1 = ln(t0 + a, g1_ref[...], be1_ref[...])
    f = mm(jax.nn.relu(mm(t1, wf1_ref[...]) + bf1_ref[...]), wf2_ref[...]) \
        + bf2_ref[...]
    t2 = ln(t1 + f, g2_ref[...], be2_ref[...])

    t2r = t2.reshape(B2, L, D)
    wr = t2r[:, 0, :]
    for j in range(1, L):
        wr = wr + t2r[:, j, :]
    wr = wr * (1.0 / L)

    slg = mm(wr, wpool_ref[...])
    slgr = slg.reshape(BN, NW, WH)
    mx2 = slgr[:, 0, :]
    for w in range(1, NW):
        mx2 = jnp.maximum(mx2, slgr[:, w, :])
    exw = [jnp.exp(slgr[:, w, :] - mx2) for w in range(NW)]
    ssum = exw[0]
    for w in range(1, NW):
        ssum = ssum + exw[w]
    alpha = [jnp.mean(exw[w] / ssum, axis=-1, keepdims=True)
             for w in range(NW)]

    wrr = wr.reshape(BN, NW, D)
    pooled = alpha[0] * wrr[:, 0, :]
    for w in range(1, NW):
        pooled = pooled + alpha[w] * wrr[:, w, :]
    out_ref[...] = jax.nn.relu(pooled)


def _walk_transformer(tok_pad, deg, w_se, Wq, Wk, Wv, Wo, g1, be1, g2, be2,
                      Wf1, bf1, Wf2, bf2, w_pool):
    BN = 80
    full = lambda shape: pl.BlockSpec(shape, lambda i: tuple(0 for _ in shape))
    return pl.pallas_call(
        _tf_body,
        grid=(N // BN,),
        in_specs=[
            pl.BlockSpec((BN * NW * L, 128), lambda i: (i, 0)),
            pl.BlockSpec((BN, 1), lambda i: (i, 0)),
            full((1, D)),
            full((D, D)), full((D, D)), full((D, D)), full((D, D)),
            full((1, D)), full((1, D)), full((1, D)), full((1, D)),
            full((D, 2 * D)), full((1, 2 * D)), full((2 * D, D)), full((1, D)),
            full((D, WH)),
        ],
        out_specs=pl.BlockSpec((BN, D), lambda i: (i, 0)),
        out_shape=jax.ShapeDtypeStruct((N, D), jnp.float32),
    )(tok_pad, deg.reshape(N, 1), w_se, Wq, Wk, Wv, Wo,
      g1.reshape(1, D), be1.reshape(1, D), g2.reshape(1, D), be2.reshape(1, D),
      Wf1, bf1.reshape(1, 2 * D), Wf2, bf2.reshape(1, D), w_pool)


@functools.partial(
    pl.kernel,
    out_type=jax.ShapeDtypeStruct((E_PAD, 128), jnp.float32),
    mesh=_MESH,
    scratch_types=[
        pltpu.VMEM((8, 128), jnp.int32),
        pltpu.VMEM((128, 128), jnp.float32),
        pltpu.SemaphoreType.DMA,
    ],
)
def _sc_gather(table_hbm, idx_hbm, out_hbm, idx_v, rows_v, sem):
    core = lax.axis_index("c")
    sub = lax.axis_index("s")
    wid = core * 16 + sub

    def chunk(ci, _):
        base = pl.multiple_of(wid * 10240 + ci * 1024, 1024)
        pltpu.sync_copy(idx_hbm.at[pl.ds(pl.multiple_of(base // 128, 8), 8)],
                        idx_v)
        for j in range(8):
            pltpu.async_copy(table_hbm.at[idx_v.at[j]], rows_v, sem).wait()
            pltpu.sync_copy(rows_v, out_hbm.at[pl.ds(base + j * 128, 128)])
        return 0

    lax.fori_loop(0, 10, chunk, 0)


@functools.partial(
    pl.kernel,
    out_type=(jax.ShapeDtypeStruct((E_PAD // 8, 128), jnp.float32),
              jax.ShapeDtypeStruct((2, NP16, 128), jnp.float32)),
    mesh=_MESH,
    scratch_types=[
        pltpu.VMEM((8, 128), jnp.int32),
        pltpu.VMEM((8, 128), jnp.int32),
        pltpu.VMEM((64, 128), jnp.float32),
        pltpu.VMEM((64, 128), jnp.float32),
        pltpu.VMEM((128, 128), jnp.float32),
        pltpu.VMEM((16, 128), jnp.float32),
        pltpu.VMEM_SHARED((NP16, 128), jnp.float32),
        pltpu.SemaphoreType.DMA,
        pltpu.SemaphoreType.DMA,
    ],
)
def _sc_edge_stats(src_hbm, dst_hbm, als_hbm, ald_hbm, ex_hbm, s_hbm,
                   srcv, dstv, asv, adv, exw, exv, s_sh, sem1, sem2):
    core = lax.axis_index("c")
    sub = lax.axis_index("s")
    wid = core * 16 + sub

    def zrow(i, _):
        for j in range(8):
            exw[i, pl.ds(j * 16, 16)] = jnp.zeros((16,), jnp.float32)
        return 0
    lax.fori_loop(0, 128, zrow, 0)
    for k in range(5):
        pltpu.sync_copy(
            exw, s_sh.at[pl.ds(pl.multiple_of(sub * 624 + k * 128, 8), 128)])
    plsc.subcore_barrier()

    def chunk(ci, _):
        base = pl.multiple_of(wid * 10240 + ci * 1024, 1024)
        rb = pl.multiple_of(base // 128, 8)
        pltpu.sync_copy(src_hbm.at[pl.ds(rb, 8)], srcv)
        pltpu.sync_copy(dst_hbm.at[pl.ds(rb, 8)], dstv)

        def sblock(sb, _):
            for hb in range(2):
                c1 = pltpu.async_copy(
                    als_hbm.at[srcv.at[sb, pl.ds(hb * 64, 64)]], asv, sem1)
                c2 = pltpu.async_copy(
                    ald_hbm.at[dstv.at[sb, pl.ds(hb * 64, 64)]], adv, sem2)
                c1.wait()
                c2.wait()
                for r in range(8):
                    for j8 in range(8):
                        e64 = r * 8 + j8
                        e = hb * 64 + e64
                        z = asv[e64, pl.ds(0, 16)] + adv[e64, pl.ds(0, 16)]
                        zl = jnp.where(z > 0, z, z * 0.2)
                        ex16 = jnp.exp(zl)
                        exw[e, pl.ds(0, 16)] = ex16
                        exv[e // 8, pl.ds((e % 8) * 16, 16)] = ex16
            pltpu.sync_copy(exw, s_sh.at[dstv.at[sb]], add=True)
            pltpu.sync_copy(
                exv,
                ex_hbm.at[pl.ds(pl.multiple_of(base // 8 + sb * 16, 8), 16)])
            return 0
        lax.fori_loop(0, 8, sblock, 0)
        return 0

    lax.fori_loop(0, 10, chunk, 0)
    plsc.subcore_barrier()
    rb0 = pl.multiple_of(sub * 624, 8)
    pltpu.sync_copy(s_sh.at[pl.ds(rb0, 640)],
                    s_hbm.at[core, pl.ds(rb0, 640)])


def _make_aggregate(wlanes):
    plan = []
    off = 0
    while off < 320:
        ln = min(128, 320 - off)
        plan.append((off, ln))
        off += ln

    @functools.partial(
        pl.kernel,
        out_type=jax.ShapeDtypeStruct((N, 128), jnp.float32),
        mesh=_MESH,
        scratch_types=[
            pltpu.VMEM((8, 128), jnp.int32),
            pltpu.VMEM((8, 128), jnp.int32),
            pltpu.VMEM((8, 128), jnp.int32),
            pltpu.VMEM((128, 128), jnp.float32),
            pltpu.VMEM((128, 128), jnp.float32),
            pltpu.VMEM((128, 128), jnp.float32),
            pltpu.VMEM((128, 128), jnp.float32),
            pltpu.VMEM((128, 128), jnp.float32),
            pltpu.VMEM_SHARED((ACC_ROWS, 128), jnp.float32),
            pltpu.SemaphoreType.DMA,
            pltpu.SemaphoreType.DMA,
        ],
    )
    def agg(src_hbm, dst_hbm, w_hbm, xp_hbm, out_hbm,
            srcv, dstv, ldv, wv_g, rows0, rows1, rows2, rows3, acc,
            sem1, sem2):
        core = lax.axis_index("c")
        sub = lax.axis_index("s")
        base_n = core * HALF
        rowbufs = [rows0, rows1, rows2, rows3]

        def zrow(i, _):
            for j in range(8):
                rows0[i, pl.ds(j * 16, 16)] = jnp.zeros((16,), jnp.float32)
            return 0
        lax.fori_loop(0, 128, zrow, 0)
        for (o, ln) in plan:
            pltpu.sync_copy(
                rows0.at[pl.ds(0, ln)],
                acc.at[pl.ds(pl.multiple_of(sub * 312 + o, 8), ln)])
        plsc.subcore_barrier()

        def group(ci, _):
            base = pl.multiple_of(sub * 20480 + ci * 1024, 1024)
            rb = pl.multiple_of(base // 128, 8)
            pltpu.sync_copy(src_hbm.at[pl.ds(rb, 8)], srcv)
            pltpu.sync_copy(dst_hbm.at[pl.ds(rb, 8)], dstv)
            pltpu.sync_copy(
                w_hbm.at[pl.ds(pl.multiple_of(base // 8, 8), 128)], wv_g)
            for g in range(64):
                row = g // 8
                colo = (g % 8) * 16
                dv = dstv[row, pl.ds(colo, 16)]
                lv = dv - base_n
                inr = (lv >= 0) & (lv < HALF)
                ldv[row, pl.ds(colo, 16)] = jnp.where(inr, lv, HALF)
            for q in range(2):
                cps = [pltpu.async_copy(xp_hbm.at[srcv.at[q * 4 + b]],
                                        rowbufs[b], sem1)
                       for b in range(4)]
                for b in range(4):
                    sbid = q * 4 + b
                    rows = rowbufs[b]
                    cps[b].wait()

                    def rbody(r, _, sbid=sbid, rows=rows):
                        for j8 in range(8):
                            e = r * 8 + j8
                            wv = wv_g[sbid * 16 + r, pl.ds(j8 * 16, 16)]
                            splats = {}
                            for j in range(8):
                                ln_ = wlanes[j]
                                if ln_ not in splats:
                                    splats[ln_] = _vsplat(wv, ln_)
                                rows[e, pl.ds(j * 16, 16)] = (
                                    rows[e, pl.ds(j * 16, 16)] * splats[ln_])
                        return 0
                    lax.fori_loop(0, 16, rbody, 0)
                    pltpu.sync_copy(rows, acc.at[ldv.at[sbid]], add=True)
            return 0

        lax.fori_loop(0, 20, group, 0)
        plsc.subcore_barrier()
        for (o, ln) in plan:
            rloc = pl.multiple_of(sub * 312 + o, 8)
            pltpu.sync_copy(
                acc.at[pl.ds(rloc, ln)],
                out_hbm.at[pl.ds(pl.multiple_of(core * HALF + rloc, 8), ln)])

    return agg


_sc_agg1a = _make_aggregate([0, 0, 1, 1, 2, 2, 3, 3])
_sc_agg1b = _make_aggregate([4, 4, 5, 5, 6, 6, 7, 7])
_sc_agg2 = _make_aggregate([0] * 8)


def _prep1_body(gt_ref, w_ref, as_ref, ad_ref, xpa_ref, xpb_ref,
                als_ref, ald_ref):
    xp = jnp.dot(gt_ref[...], w_ref[...], precision=_PREC,
                 preferred_element_type=jnp.float32)
    xpa_ref[...] = xp[:, :128]
    xpb_ref[...] = xp[:, 128:]
    als_ref[...] = jnp.dot(xp, as_ref[...], precision=_PREC,
                           preferred_element_type=jnp.float32)
    ald_ref[...] = jnp.dot(xp, ad_ref[...], precision=_PREC,
                           preferred_element_type=jnp.float32)


def _prep1(gt, W1, As, Ad):
    BN = 1000
    F = H1 * C1
    return pl.pallas_call(
        _prep1_body,
        grid=(N // BN,),
        in_specs=[
            pl.BlockSpec((BN, D), lambda i: (i, 0)),
            pl.BlockSpec((D, F), lambda i: (0, 0)),
            pl.BlockSpec((F, 128), lambda i: (0, 0)),
            pl.BlockSpec((F, 128), lambda i: (0, 0)),
        ],
        out_specs=[
            pl.BlockSpec((BN, 128), lambda i: (i, 0)),
            pl.BlockSpec((BN, 128), lambda i: (i, 0)),
            pl.BlockSpec((BN, 128), lambda i: (i, 0)),
            pl.BlockSpec((BN, 128), lambda i: (i, 0)),
        ],
        out_shape=[
            jax.ShapeDtypeStruct((N, 128), jnp.float32),
            jax.ShapeDtypeStruct((N, 128), jnp.float32),
            jax.ShapeDtypeStruct((N, 128), jnp.float32),
            jax.ShapeDtypeStruct((N, 128), jnp.float32),
        ],
    )(gt, W1, As, Ad)


def _prep2_body(o1a_ref, o1b_ref, r_ref, wa_ref, wb_ref, as_ref, ad_ref,
                xp_ref, als_ref, ald_ref):
    BN = o1a_ref.shape[0]
    li = jax.lax.broadcasted_iota(jnp.int32, (8, 128), 1)
    hi = jax.lax.broadcasted_iota(jnp.int32, (8, 128), 0)
    ea = jnp.where(li // 32 == hi, 1.0, 0.0)
    eb = jnp.where(li // 32 + 4 == hi, 1.0, 0.0)
    r8 = r_ref[...][:, :8]
    xa = jax.nn.relu(o1a_ref[...] * jnp.dot(r8, ea, precision=_PREC,
                                            preferred_element_type=jnp.float32))
    xb = jax.nn.relu(o1b_ref[...] * jnp.dot(r8, eb, precision=_PREC,
                                            preferred_element_type=jnp.float32))
    xp = (jnp.dot(xa, wa_ref[...], precision=_PREC,
                  preferred_element_type=jnp.float32)
          + jnp.dot(xb, wb_ref[...], precision=_PREC,
                    preferred_element_type=jnp.float32))
    xp_ref[...] = xp
    als_ref[...] = jnp.dot(xp, as_ref[...], precision=_PREC,
                           preferred_element_type=jnp.float32)
    ald_ref[...] = jnp.dot(xp, ad_ref[...], precision=_PREC,
                           preferred_element_type=jnp.float32)


def _prep2(o1a, o1b, r1, W2pa, W2pb, As, Ad):
    BN = 1000
    return pl.pallas_call(
        _prep2_body,
        grid=(N // BN,),
        in_specs=[
            pl.BlockSpec((BN, 128), lambda i: (i, 0)),
            pl.BlockSpec((BN, 128), lambda i: (i, 0)),
            pl.BlockSpec((BN, 128), lambda i: (i, 0)),
            pl.BlockSpec((128, 128), lambda i: (0, 0)),
            pl.BlockSpec((128, 128), lambda i: (0, 0)),
            pl.BlockSpec((128, 128), lambda i: (0, 0)),
            pl.BlockSpec((128, 128), lambda i: (0, 0)),
        ],
        out_specs=[
            pl.BlockSpec((BN, 128), lambda i: (i, 0)),
            pl.BlockSpec((BN, 128), lambda i: (i, 0)),
            pl.BlockSpec((BN, 128), lambda i: (i, 0)),
        ],
        out_shape=[
            jax.ShapeDtypeStruct((N, 128), jnp.float32),
            jax.ShapeDtypeStruct((N, 128), jnp.float32),
            jax.ShapeDtypeStruct((N, 128), jnp.float32),
        ],
    )(o1a, o1b, r1, W2pa, W2pb, As, Ad)


def _recip_body(s_ref, r_ref):
    r_ref[...] = 1.0 / (s_ref[0] + s_ref[1] + 1e-16)


def _recip(s):
    BR = 2504
    return pl.pallas_call(
        _recip_body,
        grid=(NP16 // BR,),
        in_specs=[pl.BlockSpec((2, BR, 128), lambda i: (0, i, 0))],
        out_specs=pl.BlockSpec((BR, 128), lambda i: (i, 0)),
        out_shape=jax.ShapeDtypeStruct((NP16, 128), jnp.float32),
    )(s)


def _softmax_body(x_ref, r_ref, o_ref):
    z = x_ref[...][:, :OUT] * r_ref[...][:, 0:1]
    m = jnp.max(z, axis=-1, keepdims=True)
    e = jnp.exp(z - m)
    o_ref[...] = e / jnp.sum(e, axis=-1, keepdims=True)


def _softmax16(x, r2):
    BN = 2000
    return pl.pallas_call(
        _softmax_body,
        grid=(N // BN,),
        in_specs=[
            pl.BlockSpec((BN, 128), lambda i: (i, 0)),
            pl.BlockSpec((BN, 128), lambda i: (i, 0)),
        ],
        out_specs=pl.BlockSpec((BN, OUT), lambda i: (i, 0)),
        out_shape=jax.ShapeDtypeStruct((N, OUT), jnp.float32),
    )(x, r2)


def kernel(x, edge_index, walks, deg, W_in, b_in, w_se, Wq, Wk, Wv, Wo, g1,
           be1, g2, be2, Wf1, bf1, Wf2, bf2, w_pool, W1, a_src1, a_dst1, W2,
           a_src2, a_dst2):
    src = edge_index[0]
    dst = edge_index[1]
    padE = E_PAD - E
    src2d = jnp.concatenate(
        [src, jnp.zeros((padE,), jnp.int32)]).reshape(NROW, 128)
    dst2d = jnp.concatenate(
        [dst, jnp.full((padE,), N, jnp.int32)]).reshape(NROW, 128)
    widx2d = jnp.concatenate(
        [walks.reshape(-1), jnp.zeros((padE,), jnp.int32)]).reshape(NROW, 128)

    F = H1 * C1
    W_in_p = jnp.concatenate([W_in, jnp.zeros((IN, 128 - D))], axis=1)
    b_in_p = jnp.concatenate([b_in, jnp.zeros((128 - D,))]).reshape(1, 128)
    W2p = jnp.concatenate([W2, jnp.zeros((F, 128 - OUT))], axis=1)
    W2pa = W2p[:128]
    W2pb = W2p[128:]
    As1 = jnp.zeros((F, 128), jnp.float32).at[
        jnp.arange(F), jnp.arange(F) // C1].set(a_src1.reshape(-1))
    Ad1 = jnp.zeros((F, 128), jnp.float32).at[
        jnp.arange(F), jnp.arange(F) // C1].set(a_dst1.reshape(-1))
    As2 = jnp.zeros((128, 128), jnp.float32).at[
        jnp.arange(OUT), 0].set(a_src2.reshape(-1))
    Ad2 = jnp.zeros((128, 128), jnp.float32).at[
        jnp.arange(OUT), 0].set(a_dst2.reshape(-1))
    zpad = jnp.zeros((16, 128), jnp.float32)

    xw = _project(x, W_in_p, b_in_p)
    tok_pad = _sc_gather(xw, widx2d)
    gt = _walk_transformer(tok_pad, deg, w_se, Wq, Wk, Wv, Wo, g1, be1, g2,
                           be2, Wf1, bf1, Wf2, bf2, w_pool)

    xp1a, xp1b, als1, ald1 = _prep1(gt, W1, As1, Ad1)
    ex1, s1 = _sc_edge_stats(src2d, dst2d, als1,
                             jnp.concatenate([ald1, zpad]))
    r1 = _recip(s1)
    out1a = _sc_agg1a(src2d, dst2d, ex1, xp1a)
    out1b = _sc_agg1b(src2d, dst2d, ex1, xp1b)

    xp2, als2, ald2 = _prep2(out1a, out1b, r1[:N], W2pa, W2pb, As2, Ad2)
    ex2, s2 = _sc_edge_stats(src2d, dst2d, als2,
                             jnp.concatenate([ald2, zpad]))
    r2 = _recip(s2)
    out2 = _sc_agg2(src2d, dst2d, ex2, xp2)
    return _softmax16(out2, r2[:N])


# speedup vs baseline: 1.9220x; 1.6222x over previous
import functools

import jax
import jax.numpy as jnp
from jax import lax
from jax.experimental import pallas as pl
from jax.experimental.pallas import tpu as pltpu
from jax.experimental.pallas import tpu_sc as plsc

N = 10000
E = 320000
IN = 128
D = 64
NW = 4
L = 8
TH = 4
WH = 4
H1 = 8
C1 = 32
OUT = 16

E_PAD = 327680
NROW = E_PAD // 128
NP16 = N + 16
HALF = N // 2
ACC_ROWS = HALF + 8

_PREC = jax.lax.Precision.HIGHEST
_MESH = plsc.VectorSubcoreMesh(core_axis_name="c", subcore_axis_name="s")


def _vsplat(vec, j):
    idx = jnp.full((16, 1), j, dtype=jnp.int32)
    return lax.gather(
        vec, idx,
        lax.GatherDimensionNumbers(offset_dims=(), collapsed_slice_dims=(0,),
                                   start_index_map=(0,)),
        (1,), mode=lax.GatherScatterMode.PROMISE_IN_BOUNDS)


def _proj_body(x_ref, w_ref, b_ref, o_ref):
    o_ref[...] = (
        jnp.dot(x_ref[...], w_ref[...], precision=_PREC,
                preferred_element_type=jnp.float32)
        + b_ref[...]
    )


def _project(x, W_in_p, b_in_p):
    BX = 1000
    return pl.pallas_call(
        _proj_body,
        grid=(N // BX,),
        in_specs=[
            pl.BlockSpec((BX, IN), lambda i: (i, 0)),
            pl.BlockSpec((IN, 128), lambda i: (0, 0)),
            pl.BlockSpec((1, 128), lambda i: (0, 0)),
        ],
        out_specs=pl.BlockSpec((BX, 128), lambda i: (i, 0)),
        out_shape=jax.ShapeDtypeStruct((N, 128), jnp.float32),
    )(x, W_in_p, b_in_p)


def _tf_body(tok_ref, deg_ref, wse_ref, wq_ref, wk_ref, wv_ref, wo_ref,
             g1_ref, be1_ref, g2_ref, be2_ref, wf1_ref, bf1_ref, wf2_ref,
             bf2_ref, wpool_ref, out_ref):
    R = tok_ref.shape[0]
    BN = R // (NW * L)
    B2 = BN * NW
    dh = D // TH

    def mm(a, b):
        return jnp.dot(a, b, preferred_element_type=jnp.float32)

    di = jax.lax.broadcasted_iota(jnp.int32, (D, TH), 0)
    hi = jax.lax.broadcasted_iota(jnp.int32, (D, TH), 1)
    hmask = jnp.where(di // dh == hi, 1.0, 0.0)
    r32 = jax.lax.broadcasted_iota(jnp.int32, (TH * L, TH * L), 0)
    c32 = jax.lax.broadcasted_iota(jnp.int32, (TH * L, TH * L), 1)
    perm_jh = jnp.where((r32 // TH == c32 % L) & (r32 % TH == c32 // L),
                        1.0, 0.0)

    se = deg_ref[...] * wse_ref[...]
    tok = tok_ref[...][:, :D]
    t0 = (tok.reshape(BN, NW * L, D) + se[:, None, :]).reshape(R, D)

    q = mm(t0, wq_ref[...])
    k = mm(t0, wk_ref[...])
    v = mm(t0, wv_ref[...])

    kr = k.reshape(B2, L, D)
    cols = []
    for j in range(L):
        kj = jnp.broadcast_to(kr[:, j][:, None, :], (B2, L, D)).reshape(R, D)
        cols.append(mm(q * kj, hmask))
    s32 = mm(jnp.concatenate(cols, axis=1), perm_jh)
    s32 = s32 * (1.0 / jnp.sqrt(jnp.float32(dh)))
    att_h = []
    for h in range(TH):
        sh = s32[:, h * L:(h + 1) * L]
        mx = jnp.max(sh, axis=-1, keepdims=True)
        exh = jnp.exp(sh - mx)
        att_h.append(exh / jnp.sum(exh, axis=-1, keepdims=True))

    vr = v.reshape(B2, L, D)
    acc = jnp.zeros((R, D), jnp.float32)
    for j in range(L):
        vj = jnp.broadcast_to(vr[:, j][:, None, :], (B2, L, D)).reshape(R, D)
        aj = jnp.concatenate(
            [jnp.broadcast_to(att_h[h][:, j:j + 1], (R, dh))
             for h in range(TH)], axis=1)
        acc = acc + aj * vj
    a = mm(acc, wo_ref[...])

    def ln(z, g, b):
        mu = jnp.mean(z, axis=-1, keepdims=True)
        var = jnp.mean(z * z, axis=-1, keepdims=True) - mu * mu
        return (z - mu) * jax.lax.rsqrt(var + 1e-5) * g + b

    t1 = ln(t0 + a, g1_ref[...], be1_ref[...])
    f = mm(jax.nn.relu(mm(t1, wf1_ref[...]) + bf1_ref[...]), wf2_ref[...]) \
        + bf2_ref[...]
    t2 = ln(t1 + f, g2_ref[...], be2_ref[...])

    t2r = t2.reshape(B2, L, D)
    wr = t2r[:, 0, :]
    for j in range(1, L):
        wr = wr + t2r[:, j, :]
    wr = wr * (1.0 / L)

    slg = mm(wr, wpool_ref[...])
    slgr = slg.reshape(BN, NW, WH)
    mx2 = slgr[:, 0, :]
    for w in range(1, NW):
        mx2 = jnp.maximum(mx2, slgr[:, w, :])
    exw = [jnp.exp(slgr[:, w, :] - mx2) for w in range(NW)]
    ssum = exw[0]
    for w in range(1, NW):
        ssum = ssum + exw[w]
    alpha = [jnp.mean(exw[w] / ssum, axis=-1, keepdims=True)
             for w in range(NW)]

    wrr = wr.reshape(BN, NW, D)
    pooled = alpha[0] * wrr[:, 0, :]
    for w in range(1, NW):
        pooled = pooled + alpha[w] * wrr[:, w, :]
    out_ref[...] = jax.nn.relu(pooled)


def _walk_transformer(tok_pad, deg, w_se, Wq, Wk, Wv, Wo, g1, be1, g2, be2,
                      Wf1, bf1, Wf2, bf2, w_pool):
    BN = 80
    full = lambda shape: pl.BlockSpec(shape, lambda i: tuple(0 for _ in shape))
    return pl.pallas_call(
        _tf_body,
        grid=(N // BN,),
        in_specs=[
            pl.BlockSpec((BN * NW * L, 128), lambda i: (i, 0)),
            pl.BlockSpec((BN, 1), lambda i: (i, 0)),
            full((1, D)),
            full((D, D)), full((D, D)), full((D, D)), full((D, D)),
            full((1, D)), full((1, D)), full((1, D)), full((1, D)),
            full((D, 2 * D)), full((1, 2 * D)), full((2 * D, D)), full((1, D)),
            full((D, WH)),
        ],
        out_specs=pl.BlockSpec((BN, D), lambda i: (i, 0)),
        out_shape=jax.ShapeDtypeStruct((N, D), jnp.float32),
    )(tok_pad, deg.reshape(N, 1), w_se, Wq, Wk, Wv, Wo,
      g1.reshape(1, D), be1.reshape(1, D), g2.reshape(1, D), be2.reshape(1, D),
      Wf1, bf1.reshape(1, 2 * D), Wf2, bf2.reshape(1, D), w_pool)


@functools.partial(
    pl.kernel,
    out_type=jax.ShapeDtypeStruct((E_PAD, 128), jnp.float32),
    mesh=_MESH,
    scratch_types=[
        pltpu.VMEM((8, 128), jnp.int32),
        pltpu.VMEM((128, 128), jnp.float32),
        pltpu.SemaphoreType.DMA,
    ],
)
def _sc_gather(table_hbm, idx_hbm, out_hbm, idx_v, rows_v, sem):
    core = lax.axis_index("c")
    sub = lax.axis_index("s")
    wid = core * 16 + sub

    def chunk(ci, _):
        base = pl.multiple_of(wid * 10240 + ci * 1024, 1024)
        pltpu.sync_copy(idx_hbm.at[pl.ds(pl.multiple_of(base // 128, 8), 8)],
                        idx_v)
        for j in range(8):
            pltpu.async_copy(table_hbm.at[idx_v.at[j]], rows_v, sem).wait()
            pltpu.sync_copy(rows_v, out_hbm.at[pl.ds(base + j * 128, 128)])
        return 0

    lax.fori_loop(0, 10, chunk, 0)


@functools.partial(
    pl.kernel,
    out_type=(jax.ShapeDtypeStruct((E_PAD // 8, 128), jnp.float32),
              jax.ShapeDtypeStruct((2, NP16, 128), jnp.float32)),
    mesh=_MESH,
    scratch_types=[
        pltpu.VMEM((8, 128), jnp.int32),
        pltpu.VMEM((8, 128), jnp.int32),
        pltpu.VMEM((64, 128), jnp.float32),
        pltpu.VMEM((64, 128), jnp.float32),
        pltpu.VMEM((128, 128), jnp.float32),
        pltpu.VMEM((16, 128), jnp.float32),
        pltpu.VMEM_SHARED((NP16, 128), jnp.float32),
        pltpu.SemaphoreType.DMA,
        pltpu.SemaphoreType.DMA,
    ],
)
def _sc_edge_stats(src_hbm, dst_hbm, als_hbm, ald_hbm, ex_hbm, s_hbm,
                   srcv, dstv, asv, adv, exw, exv, s_sh, sem1, sem2):
    core = lax.axis_index("c")
    sub = lax.axis_index("s")
    wid = core * 16 + sub

    def zrow(i, _):
        for j in range(8):
            exw[i, pl.ds(j * 16, 16)] = jnp.zeros((16,), jnp.float32)
        return 0
    lax.fori_loop(0, 128, zrow, 0)
    for k in range(5):
        pltpu.sync_copy(
            exw, s_sh.at[pl.ds(pl.multiple_of(sub * 624 + k * 128, 8), 128)])
    plsc.subcore_barrier()

    def chunk(ci, _):
        base = pl.multiple_of(wid * 10240 + ci * 1024, 1024)
        rb = pl.multiple_of(base // 128, 8)
        pltpu.sync_copy(src_hbm.at[pl.ds(rb, 8)], srcv)
        pltpu.sync_copy(dst_hbm.at[pl.ds(rb, 8)], dstv)

        def sblock(sb, _):
            for hb in range(2):
                c1 = pltpu.async_copy(
                    als_hbm.at[srcv.at[sb, pl.ds(hb * 64, 64)]], asv, sem1)
                c2 = pltpu.async_copy(
                    ald_hbm.at[dstv.at[sb, pl.ds(hb * 64, 64)]], adv, sem2)
                c1.wait()
                c2.wait()
                for r in range(8):
                    for j8 in range(8):
                        e64 = r * 8 + j8
                        e = hb * 64 + e64
                        z = asv[e64, pl.ds(0, 16)] + adv[e64, pl.ds(0, 16)]
                        zl = jnp.where(z > 0, z, z * 0.2)
                        ex16 = jnp.exp(zl)
                        exw[e, pl.ds(0, 16)] = ex16
                        exv[e // 8, pl.ds((e % 8) * 16, 16)] = ex16
            pltpu.sync_copy(exw, s_sh.at[dstv.at[sb]], add=True)
            pltpu.sync_copy(
                exv,
                ex_hbm.at[pl.ds(pl.multiple_of(base // 8 + sb * 16, 8), 16)])
            return 0
        lax.fori_loop(0, 8, sblock, 0)
        return 0

    lax.fori_loop(0, 10, chunk, 0)
    plsc.subcore_barrier()
    rb0 = pl.multiple_of(sub * 624, 8)
    pltpu.sync_copy(s_sh.at[pl.ds(rb0, 640)],
                    s_hbm.at[core, pl.ds(rb0, 640)])


def _make_aggregate(wlanes):
    plan = []
    off = 0
    while off < 320:
        ln = min(128, 320 - off)
        plan.append((off, ln))
        off += ln

    @functools.partial(
        pl.kernel,
        out_type=jax.ShapeDtypeStruct((N, 128), jnp.float32),
        mesh=_MESH,
        scratch_types=[
            pltpu.VMEM((8, 128), jnp.int32),
            pltpu.VMEM((8, 128), jnp.int32),
            pltpu.VMEM((8, 128), jnp.int32),
            pltpu.VMEM((128, 128), jnp.float32),
            pltpu.VMEM((128, 128), jnp.float32),
            pltpu.VMEM((128, 128), jnp.float32),
            pltpu.VMEM((128, 128), jnp.float32),
            pltpu.VMEM((128, 128), jnp.float32),
            pltpu.VMEM_SHARED((ACC_ROWS, 128), jnp.float32),
            pltpu.SemaphoreType.DMA,
            pltpu.SemaphoreType.DMA,
        ],
    )
    def agg(src_hbm, dst_hbm, w_hbm, xp_hbm, out_hbm,
            srcv, dstv, ldv, wv_g, rows0, rows1, rows2, rows3, acc,
            sem1, sem2):
        core = lax.axis_index("c")
        sub = lax.axis_index("s")
        base_n = core * HALF
        rowbufs = [rows0, rows1, rows2, rows3]

        def zrow(i, _):
            for j in range(8):
                rows0[i, pl.ds(j * 16, 16)] = jnp.zeros((16,), jnp.float32)
            return 0
        lax.fori_loop(0, 128, zrow, 0)
        for (o, ln) in plan:
            pltpu.sync_copy(
                rows0.at[pl.ds(0, ln)],
                acc.at[pl.ds(pl.multiple_of(sub * 312 + o, 8), ln)])
        plsc.subcore_barrier()

        def group(ci, _):
            base = pl.multiple_of(sub * 20480 + ci * 1024, 1024)
            rb = pl.multiple_of(base // 128, 8)
            pltpu.sync_copy(src_hbm.at[pl.ds(rb, 8)], srcv)
            pltpu.sync_copy(dst_hbm.at[pl.ds(rb, 8)], dstv)
            pltpu.sync_copy(
                w_hbm.at[pl.ds(pl.multiple_of(base // 8, 8), 128)], wv_g)
            for g in range(64):
                row = g // 8
                colo = (g % 8) * 16
                dv = dstv[row, pl.ds(colo, 16)]
                lv = dv - base_n
                inr = (lv >= 0) & (lv < HALF)
                ldv[row, pl.ds(colo, 16)] = jnp.where(inr, lv, HALF)
            for q in range(2):
                cps = [pltpu.async_copy(xp_hbm.at[srcv.at[q * 4 + b]],
                                        rowbufs[b], sem1)
                       for b in range(4)]
                for b in range(4):
                    sbid = q * 4 + b
                    rows = rowbufs[b]
                    cps[b].wait()

                    def rbody(r, _, sbid=sbid, rows=rows):
                        for j8 in range(8):
                            e = r * 8 + j8
                            wv = wv_g[sbid * 16 + r, pl.ds(j8 * 16, 16)]
                            splats = {}
                            for j in range(8):
                                ln_ = wlanes[j]
                                if ln_ not in splats:
                                    splats[ln_] = _vsplat(wv, ln_)
                                rows[e, pl.ds(j * 16, 16)] = (
                                    rows[e, pl.ds(j * 16, 16)] * splats[ln_])
                        return 0
                    lax.fori_loop(0, 16, rbody, 0)
                    pltpu.sync_copy(rows, acc.at[ldv.at[sbid]], add=True)
            return 0

        lax.fori_loop(0, 20, group, 0)
        plsc.subcore_barrier()
        for (o, ln) in plan:
            rloc = pl.multiple_of(sub * 312 + o, 8)
            pltpu.sync_copy(
                acc.at[pl.ds(rloc, ln)],
                out_hbm.at[pl.ds(pl.multiple_of(core * HALF + rloc, 8), ln)])

    return agg


_sc_agg1a = _make_aggregate([0, 0, 1, 1, 2, 2, 3, 3])
_sc_agg1b = _make_aggregate([4, 4, 5, 5, 6, 6, 7, 7])
_sc_agg2 = _make_aggregate([0] * 8)


def _prep1_body(gt_ref, w_ref, as_ref, ad_ref, xpa_ref, xpb_ref,
                als_ref, ald_ref):
    xp = jnp.dot(gt_ref[...], w_ref[...], precision=_PREC,
                 preferred_element_type=jnp.float32)
    xpa_ref[...] = xp[:, :128]
    xpb_ref[...] = xp[:, 128:]
    als_ref[...] = jnp.dot(xp, as_ref[...], precision=_PREC,
                           preferred_element_type=jnp.float32)
    ald_ref[...] = jnp.dot(xp, ad_ref[...], precision=_PREC,
                           preferred_element_type=jnp.float32)


def _prep1(gt, W1, As, Ad):
    BN = 1000
    F = H1 * C1
    return pl.pallas_call(
        _prep1_body,
        grid=(N // BN,),
        in_specs=[
            pl.BlockSpec((BN, D), lambda i: (i, 0)),
            pl.BlockSpec((D, F), lambda i: (0, 0)),
            pl.BlockSpec((F, 128), lambda i: (0, 0)),
            pl.BlockSpec((F, 128), lambda i: (0, 0)),
        ],
        out_specs=[
            pl.BlockSpec((BN, 128), lambda i: (i, 0)),
            pl.BlockSpec((BN, 128), lambda i: (i, 0)),
            pl.BlockSpec((BN, 128), lambda i: (i, 0)),
            pl.BlockSpec((BN, 128), lambda i: (i, 0)),
        ],
        out_shape=[
            jax.ShapeDtypeStruct((N, 128), jnp.float32),
            jax.ShapeDtypeStruct((N, 128), jnp.float32),
            jax.ShapeDtypeStruct((N, 128), jnp.float32),
            jax.ShapeDtypeStruct((N, 128), jnp.float32),
        ],
    )(gt, W1, As, Ad)


def _prep2_body(o1a_ref, o1b_ref, r_ref, wa_ref, wb_ref, as_ref, ad_ref,
                xp_ref, als_ref, ald_ref):
    BN = o1a_ref.shape[0]
    li = jax.lax.broadcasted_iota(jnp.int32, (8, 128), 1)
    hi = jax.lax.broadcasted_iota(jnp.int32, (8, 128), 0)
    ea = jnp.where(li // 32 == hi, 1.0, 0.0)
    eb = jnp.where(li // 32 + 4 == hi, 1.0, 0.0)
    r8 = r_ref[...][:, :8]
    xa = jax.nn.relu(o1a_ref[...] * jnp.dot(r8, ea, precision=_PREC,
                                            preferred_element_type=jnp.float32))
    xb = jax.nn.relu(o1b_ref[...] * jnp.dot(r8, eb, precision=_PREC,
                                            preferred_element_type=jnp.float32))
    xp = (jnp.dot(xa, wa_ref[...], precision=_PREC,
                  preferred_element_type=jnp.float32)
          + jnp.dot(xb, wb_ref[...], precision=_PREC,
                    preferred_element_type=jnp.float32))
    xp_ref[...] = xp
    als_ref[...] = jnp.dot(xp, as_ref[...], precision=_PREC,
                           preferred_element_type=jnp.float32)
    ald_ref[...] = jnp.dot(xp, ad_ref[...], precision=_PREC,
                           preferred_element_type=jnp.float32)


def _prep2(o1a, o1b, r1, W2pa, W2pb, As, Ad):
    BN = 1000
    return pl.pallas_call(
        _prep2_body,
        grid=(N // BN,),
        in_specs=[
            pl.BlockSpec((BN, 128), lambda i: (i, 0)),
            pl.BlockSpec((BN, 128), lambda i: (i, 0)),
            pl.BlockSpec((BN, 128), lambda i: (i, 0)),
            pl.BlockSpec((128, 128), lambda i: (0, 0)),
            pl.BlockSpec((128, 128), lambda i: (0, 0)),
            pl.BlockSpec((128, 128), lambda i: (0, 0)),
            pl.BlockSpec((128, 128), lambda i: (0, 0)),
        ],
        out_specs=[
            pl.BlockSpec((BN, 128), lambda i: (i, 0)),
            pl.BlockSpec((BN, 128), lambda i: (i, 0)),
            pl.BlockSpec((BN, 128), lambda i: (i, 0)),
        ],
        out_shape=[
            jax.ShapeDtypeStruct((N, 128), jnp.float32),
            jax.ShapeDtypeStruct((N, 128), jnp.float32),
            jax.ShapeDtypeStruct((N, 128), jnp.float32),
        ],
    )(o1a, o1b, r1, W2pa, W2pb, As, Ad)


def _recip_body(s_ref, r_ref):
    r_ref[...] = 1.0 / (s_ref[0] + s_ref[1] + 1e-16)


def _recip(s):
    BR = 2504
    return pl.pallas_call(
        _recip_body,
        grid=(NP16 // BR,),
        in_specs=[pl.BlockSpec((2, BR, 128), lambda i: (0, i, 0))],
        out_specs=pl.BlockSpec((BR, 128), lambda i: (i, 0)),
        out_shape=jax.ShapeDtypeStruct((NP16, 128), jnp.float32),
    )(s)


def _softmax_body(x_ref, r_ref, o_ref):
    z = x_ref[...][:, :OUT] * r_ref[...][:, 0:1]
    m = jnp.max(z, axis=-1, keepdims=True)
    e = jnp.exp(z - m)
    o_ref[...] = e / jnp.sum(e, axis=-1, keepdims=True)


def _softmax16(x, r2):
    BN = 2000
    return pl.pallas_call(
        _softmax_body,
        grid=(N // BN,),
        in_specs=[
            pl.BlockSpec((BN, 128), lambda i: (i, 0)),
            pl.BlockSpec((BN, 128), lambda i: (i, 0)),
        ],
        out_specs=pl.BlockSpec((BN, OUT), lambda i: (i, 0)),
        out_shape=jax.ShapeDtypeStruct((N, OUT), jnp.float32),
    )(x, r2)


def kernel(x, edge_index, walks, deg, W_in, b_in, w_se, Wq, Wk, Wv, Wo, g1,
           be1, g2, be2, Wf1, bf1, Wf2, bf2, w_pool, W1, a_src1, a_dst1, W2,
           a_src2, a_dst2):
    src = edge_index[0]
    dst = edge_index[1]
    padE = E_PAD - E
    src2d = jnp.concatenate(
        [src, jnp.zeros((padE,), jnp.int32)]).reshape(NROW, 128)
    dst2d = jnp.concatenate(
        [dst, jnp.full((padE,), N, jnp.int32)]).reshape(NROW, 128)
    widx2d = jnp.concatenate(
        [walks.reshape(-1), jnp.zeros((padE,), jnp.int32)]).reshape(NROW, 128)

    F = H1 * C1
    W_in_p = jnp.concatenate([W_in, jnp.zeros((IN, 128 - D))], axis=1)
    b_in_p = jnp.concatenate([b_in, jnp.zeros((128 - D,))]).reshape(1, 128)
    W2p = jnp.concatenate([W2, jnp.zeros((F, 128 - OUT))], axis=1)
    W2pa = W2p[:128]
    W2pb = W2p[128:]
    As1 = jnp.zeros((F, 128), jnp.float32).at[
        jnp.arange(F), jnp.arange(F) // C1].set(a_src1.reshape(-1))
    Ad1 = jnp.zeros((F, 128), jnp.float32).at[
        jnp.arange(F), jnp.arange(F) // C1].set(a_dst1.reshape(-1))
    As2 = jnp.zeros((128, 128), jnp.float32).at[
        jnp.arange(OUT), 0].set(a_src2.reshape(-1))
    Ad2 = jnp.zeros((128, 128), jnp.float32).at[
        jnp.arange(OUT), 0].set(a_dst2.reshape(-1))
    zpad = jnp.zeros((16, 128), jnp.float32)

    xw = _project(x, W_in_p, b_in_p)
    tok_pad = _sc_gather(xw, widx2d)
    gt = _walk_transformer(tok_pad, deg, w_se, Wq, Wk, Wv, Wo, g1, be1, g2,
                           be2, Wf1, bf1, Wf2, bf2, w_pool)

    xp1a, xp1b, als1, ald1 = _prep1(gt, W1, As1, Ad1)
    ex1, s1 = _sc_edge_stats(src2d, dst2d, als1,
                             jnp.concatenate([ald1, zpad]))
    r1 = _recip(s1)
    out1a = _sc_agg1a(src2d, dst2d, ex1, xp1a)
    out1b = _sc_agg1b(src2d, dst2d, ex1, xp1b)

    xp2, als2, ald2 = _prep2(out1a, out1b, r1[:N], W2pa, W2pb, As2, Ad2)
    ex2, s2 = _sc_edge_stats(src2d, dst2d, als2,
                             jnp.concatenate([ald2, zpad]))
    r2 = _recip(s2)
    out2 = _sc_agg2(src2d, dst2d, ex2, xp2)
    return _softmax16(out2, r2[:N])


# speedup vs baseline: 1.9427x; 1.0107x over previous
import functools

import jax
import jax.numpy as jnp
from jax import lax
from jax.experimental import pallas as pl
from jax.experimental.pallas import tpu as pltpu
from jax.experimental.pallas import tpu_sc as plsc

N = 10000
E = 320000
IN = 128
D = 64
NW = 4
L = 8
TH = 4
WH = 4
H1 = 8
C1 = 32
OUT = 16

E_PAD = 327680
NROW = E_PAD // 128
NP16 = N + 16
HALF = N // 2
ACC_ROWS = HALF + 8

_MESH = plsc.VectorSubcoreMesh(core_axis_name="c", subcore_axis_name="s")


def _vsplat(vec, j):
    idx = jnp.full((16, 1), j, dtype=jnp.int32)
    return lax.gather(
        vec, idx,
        lax.GatherDimensionNumbers(offset_dims=(), collapsed_slice_dims=(0,),
                                   start_index_map=(0,)),
        (1,), mode=lax.GatherScatterMode.PROMISE_IN_BOUNDS)


def _proj_body(x_ref, w_ref, b_ref, o_ref):
    o_ref[...] = (
        jnp.dot(x_ref[...], w_ref[...], preferred_element_type=jnp.float32)
        + b_ref[...]
    )


def _project(x, W_in_p, b_in_p):
    BX = 1000
    return pl.pallas_call(
        _proj_body,
        grid=(N // BX,),
        in_specs=[
            pl.BlockSpec((BX, IN), lambda i: (i, 0)),
            pl.BlockSpec((IN, 128), lambda i: (0, 0)),
            pl.BlockSpec((1, 128), lambda i: (0, 0)),
        ],
        out_specs=pl.BlockSpec((BX, 128), lambda i: (i, 0)),
        out_shape=jax.ShapeDtypeStruct((N, 128), jnp.float32),
    )(x, W_in_p, b_in_p)


def _tf_body(tok_ref, deg_ref, wse_ref, wq_ref, wk_ref, wv_ref, wo_ref,
             g1_ref, be1_ref, g2_ref, be2_ref, wf1_ref, bf1_ref, wf2_ref,
             bf2_ref, wpool_ref, out_ref):
    R = tok_ref.shape[0]
    BN = R // (NW * L)
    B2 = BN * NW
    dh = D // TH

    def mm(a, b):
        return jnp.dot(a, b, preferred_element_type=jnp.float32)

    di = jax.lax.broadcasted_iota(jnp.int32, (D, TH), 0)
    hi = jax.lax.broadcasted_iota(jnp.int32, (D, TH), 1)
    hmask = jnp.where(di // dh == hi, 1.0, 0.0)
    r32 = jax.lax.broadcasted_iota(jnp.int32, (TH * L, TH * L), 0)
    c32 = jax.lax.broadcasted_iota(jnp.int32, (TH * L, TH * L), 1)
    perm_jh = jnp.where((r32 // TH == c32 % L) & (r32 % TH == c32 // L),
                        1.0, 0.0)

    se = deg_ref[...] * wse_ref[...]
    tok = tok_ref[...][:, :D]
    t0 = (tok.reshape(BN, NW * L, D) + se[:, None, :]).reshape(R, D)

    q = mm(t0, wq_ref[...])
    k = mm(t0, wk_ref[...])
    v = mm(t0, wv_ref[...])

    kr = k.reshape(B2, L, D)
    cols = []
    for j in range(L):
        kj = jnp.broadcast_to(kr[:, j][:, None, :], (B2, L, D)).reshape(R, D)
        cols.append(mm(q * kj, hmask))
    s32 = mm(jnp.concatenate(cols, axis=1), perm_jh)
    s32 = s32 * (1.0 / jnp.sqrt(jnp.float32(dh)))
    att_h = []
    for h in range(TH):
        sh = s32[:, h * L:(h + 1) * L]
        mx = jnp.max(sh, axis=-1, keepdims=True)
        exh = jnp.exp(sh - mx)
        att_h.append(exh / jnp.sum(exh, axis=-1, keepdims=True))

    vr = v.reshape(B2, L, D)
    acc = jnp.zeros((R, D), jnp.float32)
    for j in range(L):
        vj = jnp.broadcast_to(vr[:, j][:, None, :], (B2, L, D)).reshape(R, D)
        aj = jnp.concatenate(
            [jnp.broadcast_to(att_h[h][:, j:j + 1], (R, dh))
             for h in range(TH)], axis=1)
        acc = acc + aj * vj
    a = mm(acc, wo_ref[...])

    def ln(z, g, b):
        mu = jnp.mean(z, axis=-1, keepdims=True)
        var = jnp.mean(z * z, axis=-1, keepdims=True) - mu * mu
        return (z - mu) * jax.lax.rsqrt(var + 1e-5) * g + b

    t1 = ln(t0 + a, g1_ref[...], be1_ref[...])
    f = mm(jax.nn.relu(mm(t1, wf1_ref[...]) + bf1_ref[...]), wf2_ref[...]) \
        + bf2_ref[...]
    t2 = ln(t1 + f, g2_ref[...], be2_ref[...])

    t2r = t2.reshape(B2, L, D)
    wr = t2r[:, 0, :]
    for j in range(1, L):
        wr = wr + t2r[:, j, :]
    wr = wr * (1.0 / L)

    slg = mm(wr, wpool_ref[...])
    slgr = slg.reshape(BN, NW, WH)
    mx2 = slgr[:, 0, :]
    for w in range(1, NW):
        mx2 = jnp.maximum(mx2, slgr[:, w, :])
    exw = [jnp.exp(slgr[:, w, :] - mx2) for w in range(NW)]
    ssum = exw[0]
    for w in range(1, NW):
        ssum = ssum + exw[w]
    alpha = [jnp.mean(exw[w] / ssum, axis=-1, keepdims=True)
             for w in range(NW)]

    wrr = wr.reshape(BN, NW, D)
    pooled = alpha[0] * wrr[:, 0, :]
    for w in range(1, NW):
        pooled = pooled + alpha[w] * wrr[:, w, :]
    out_ref[...] = jax.nn.relu(pooled)


def _walk_transformer(tok_pad, deg, w_se, Wq, Wk, Wv, Wo, g1, be1, g2, be2,
                      Wf1, bf1, Wf2, bf2, w_pool):
    BN = 80
    full = lambda shape: pl.BlockSpec(shape, lambda i: tuple(0 for _ in shape))
    return pl.pallas_call(
        _tf_body,
        grid=(N // BN,),
        in_specs=[
            pl.BlockSpec((BN * NW * L, 128), lambda i: (i, 0)),
            pl.BlockSpec((BN, 1), lambda i: (i, 0)),
            full((1, D)),
            full((D, D)), full((D, D)), full((D, D)), full((D, D)),
            full((1, D)), full((1, D)), full((1, D)), full((1, D)),
            full((D, 2 * D)), full((1, 2 * D)), full((2 * D, D)), full((1, D)),
            full((D, WH)),
        ],
        out_specs=pl.BlockSpec((BN, D), lambda i: (i, 0)),
        out_shape=jax.ShapeDtypeStruct((N, D), jnp.float32),
    )(tok_pad, deg.reshape(N, 1), w_se, Wq, Wk, Wv, Wo,
      g1.reshape(1, D), be1.reshape(1, D), g2.reshape(1, D), be2.reshape(1, D),
      Wf1, bf1.reshape(1, 2 * D), Wf2, bf2.reshape(1, D), w_pool)


@functools.partial(
    pl.kernel,
    out_type=jax.ShapeDtypeStruct((E_PAD, 128), jnp.float32),
    mesh=_MESH,
    scratch_types=[
        pltpu.VMEM((8, 128), jnp.int32),
        pltpu.VMEM((128, 128), jnp.float32),
        pltpu.SemaphoreType.DMA,
    ],
)
def _sc_gather(table_hbm, idx_hbm, out_hbm, idx_v, rows_v, sem):
    core = lax.axis_index("c")
    sub = lax.axis_index("s")
    wid = core * 16 + sub

    def chunk(ci, _):
        base = pl.multiple_of(wid * 10240 + ci * 1024, 1024)
        pltpu.sync_copy(idx_hbm.at[pl.ds(pl.multiple_of(base // 128, 8), 8)],
                        idx_v)
        for j in range(8):
            pltpu.async_copy(table_hbm.at[idx_v.at[j]], rows_v, sem).wait()
            pltpu.sync_copy(rows_v, out_hbm.at[pl.ds(base + j * 128, 128)])
        return 0

    lax.fori_loop(0, 10, chunk, 0)


@functools.partial(
    pl.kernel,
    out_type=(jax.ShapeDtypeStruct((E_PAD // 8, 128), jnp.float32),
              jax.ShapeDtypeStruct((2, NP16, 128), jnp.float32)),
    mesh=_MESH,
    scratch_types=[
        pltpu.VMEM((8, 128), jnp.int32),
        pltpu.VMEM((8, 128), jnp.int32),
        pltpu.VMEM((64, 128), jnp.float32),
        pltpu.VMEM((64, 128), jnp.float32),
        pltpu.VMEM((128, 128), jnp.float32),
        pltpu.VMEM((16, 128), jnp.float32),
        pltpu.VMEM_SHARED((NP16, 128), jnp.float32),
        pltpu.SemaphoreType.DMA,
        pltpu.SemaphoreType.DMA,
    ],
)
def _sc_edge_stats(src_hbm, dst_hbm, als_hbm, ald_hbm, ex_hbm, s_hbm,
                   srcv, dstv, asv, adv, exw, exv, s_sh, sem1, sem2):
    core = lax.axis_index("c")
    sub = lax.axis_index("s")
    wid = core * 16 + sub

    def zrow(i, _):
        for j in range(8):
            exw[i, pl.ds(j * 16, 16)] = jnp.zeros((16,), jnp.float32)
        return 0
    lax.fori_loop(0, 128, zrow, 0)
    for k in range(5):
        pltpu.sync_copy(
            exw, s_sh.at[pl.ds(pl.multiple_of(sub * 624 + k * 128, 8), 128)])
    plsc.subcore_barrier()

    def chunk(ci, _):
        base = pl.multiple_of(wid * 10240 + ci * 1024, 1024)
        rb = pl.multiple_of(base // 128, 8)
        pltpu.sync_copy(src_hbm.at[pl.ds(rb, 8)], srcv)
        pltpu.sync_copy(dst_hbm.at[pl.ds(rb, 8)], dstv)

        def sblock(sb, _):
            for hb in range(2):
                c1 = pltpu.async_copy(
                    als_hbm.at[srcv.at[sb, pl.ds(hb * 64, 64)]], asv, sem1)
                c2 = pltpu.async_copy(
                    ald_hbm.at[dstv.at[sb, pl.ds(hb * 64, 64)]], adv, sem2)
                c1.wait()
                c2.wait()
                for r in range(8):
                    for j8 in range(8):
                        e64 = r * 8 + j8
                        e = hb * 64 + e64
                        z = asv[e64, pl.ds(0, 16)] + adv[e64, pl.ds(0, 16)]
                        zl = jnp.where(z > 0, z, z * 0.2)
                        ex16 = jnp.exp(zl)
                        exw[e, pl.ds(0, 16)] = ex16
                        exv[e // 8, pl.ds((e % 8) * 16, 16)] = ex16
            pltpu.sync_copy(exw, s_sh.at[dstv.at[sb]], add=True)
            pltpu.sync_copy(
                exv,
                ex_hbm.at[pl.ds(pl.multiple_of(base // 8 + sb * 16, 8), 16)])
            return 0
        lax.fori_loop(0, 8, sblock, 0)
        return 0

    lax.fori_loop(0, 10, chunk, 0)
    plsc.subcore_barrier()
    rb0 = pl.multiple_of(sub * 624, 8)
    pltpu.sync_copy(s_sh.at[pl.ds(rb0, 640)],
                    s_hbm.at[core, pl.ds(rb0, 640)])


def _make_aggregate(wlanes):
    plan = []
    off = 0
    while off < 320:
        ln = min(128, 320 - off)
        plan.append((off, ln))
        off += ln

    @functools.partial(
        pl.kernel,
        out_type=jax.ShapeDtypeStruct((N, 128), jnp.float32),
        mesh=_MESH,
        scratch_types=[
            pltpu.VMEM((8, 128), jnp.int32),
            pltpu.VMEM((8, 128), jnp.int32),
            pltpu.VMEM((8, 128), jnp.int32),
            pltpu.VMEM((128, 128), jnp.float32),
            pltpu.VMEM((128, 128), jnp.float32),
            pltpu.VMEM((128, 128), jnp.float32),
            pltpu.VMEM((128, 128), jnp.float32),
            pltpu.VMEM((128, 128), jnp.float32),
            pltpu.VMEM_SHARED((ACC_ROWS, 128), jnp.float32),
            pltpu.SemaphoreType.DMA,
            pltpu.SemaphoreType.DMA,
        ],
    )
    def agg(src_hbm, dst_hbm, w_hbm, xp_hbm, out_hbm,
            srcv, dstv, ldv, wv_g, rows0, rows1, rows2, rows3, acc,
            sem1, sem2):
        core = lax.axis_index("c")
        sub = lax.axis_index("s")
        base_n = core * HALF
        rowbufs = [rows0, rows1, rows2, rows3]

        def zrow(i, _):
            for j in range(8):
                rows0[i, pl.ds(j * 16, 16)] = jnp.zeros((16,), jnp.float32)
            return 0
        lax.fori_loop(0, 128, zrow, 0)
        for (o, ln) in plan:
            pltpu.sync_copy(
                rows0.at[pl.ds(0, ln)],
                acc.at[pl.ds(pl.multiple_of(sub * 312 + o, 8), ln)])
        plsc.subcore_barrier()

        def group(ci, _):
            base = pl.multiple_of(sub * 20480 + ci * 1024, 1024)
            rb = pl.multiple_of(base // 128, 8)
            pltpu.sync_copy(src_hbm.at[pl.ds(rb, 8)], srcv)
            pltpu.sync_copy(dst_hbm.at[pl.ds(rb, 8)], dstv)
            pltpu.sync_copy(
                w_hbm.at[pl.ds(pl.multiple_of(base // 8, 8), 128)], wv_g)
            for g in range(64):
                row = g // 8
                colo = (g % 8) * 16
                dv = dstv[row, pl.ds(colo, 16)]
                lv = dv - base_n
                inr = (lv >= 0) & (lv < HALF)
                ldv[row, pl.ds(colo, 16)] = jnp.where(inr, lv, HALF)
            for q in range(2):
                cps = [pltpu.async_copy(xp_hbm.at[srcv.at[q * 4 + b]],
                                        rowbufs[b], sem1)
                       for b in range(4)]
                for b in range(4):
                    sbid = q * 4 + b
                    rows = rowbufs[b]
                    cps[b].wait()

                    def rbody(r, _, sbid=sbid, rows=rows):
                        for j8 in range(8):
                            e = r * 8 + j8
                            wv = wv_g[sbid * 16 + r, pl.ds(j8 * 16, 16)]
                            splats = {}
                            for j in range(8):
                                ln_ = wlanes[j]
                                if ln_ not in splats:
                                    splats[ln_] = _vsplat(wv, ln_)
                                rows[e, pl.ds(j * 16, 16)] = (
                                    rows[e, pl.ds(j * 16, 16)] * splats[ln_])
                        return 0
                    lax.fori_loop(0, 16, rbody, 0)
                    pltpu.sync_copy(rows, acc.at[ldv.at[sbid]], add=True)
            return 0

        lax.fori_loop(0, 20, group, 0)
        plsc.subcore_barrier()
        for (o, ln) in plan:
            rloc = pl.multiple_of(sub * 312 + o, 8)
            pltpu.sync_copy(
                acc.at[pl.ds(rloc, ln)],
                out_hbm.at[pl.ds(pl.multiple_of(core * HALF + rloc, 8), ln)])

    return agg


_sc_agg1a = _make_aggregate([0, 0, 1, 1, 2, 2, 3, 3])
_sc_agg1b = _make_aggregate([4, 4, 5, 5, 6, 6, 7, 7])
_sc_agg2 = _make_aggregate([0] * 8)


def _prep1_body(gt_ref, w_ref, as_ref, ad_ref, xpa_ref, xpb_ref,
                als_ref, ald_ref):
    xp = jnp.dot(gt_ref[...], w_ref[...],                  preferred_element_type=jnp.float32)
    xpa_ref[...] = xp[:, :128]
    xpb_ref[...] = xp[:, 128:]
    als_ref[...] = jnp.dot(xp, as_ref[...],                            preferred_element_type=jnp.float32)
    ald_ref[...] = jnp.dot(xp, ad_ref[...],                            preferred_element_type=jnp.float32)


def _prep1(gt, W1, As, Ad):
    BN = 1000
    F = H1 * C1
    return pl.pallas_call(
        _prep1_body,
        grid=(N // BN,),
        in_specs=[
            pl.BlockSpec((BN, D), lambda i: (i, 0)),
            pl.BlockSpec((D, F), lambda i: (0, 0)),
            pl.BlockSpec((F, 128), lambda i: (0, 0)),
            pl.BlockSpec((F, 128), lambda i: (0, 0)),
        ],
        out_specs=[
            pl.BlockSpec((BN, 128), lambda i: (i, 0)),
            pl.BlockSpec((BN, 128), lambda i: (i, 0)),
            pl.BlockSpec((BN, 128), lambda i: (i, 0)),
            pl.BlockSpec((BN, 128), lambda i: (i, 0)),
        ],
        out_shape=[
            jax.ShapeDtypeStruct((N, 128), jnp.float32),
            jax.ShapeDtypeStruct((N, 128), jnp.float32),
            jax.ShapeDtypeStruct((N, 128), jnp.float32),
            jax.ShapeDtypeStruct((N, 128), jnp.float32),
        ],
    )(gt, W1, As, Ad)


def _prep2_body(o1a_ref, o1b_ref, r_ref, wa_ref, wb_ref, as_ref, ad_ref,
                xp_ref, als_ref, ald_ref):
    BN = o1a_ref.shape[0]
    li = jax.lax.broadcasted_iota(jnp.int32, (8, 128), 1)
    hi = jax.lax.broadcasted_iota(jnp.int32, (8, 128), 0)
    ea = jnp.where(li // 32 == hi, 1.0, 0.0)
    eb = jnp.where(li // 32 + 4 == hi, 1.0, 0.0)
    r8 = r_ref[...][:, :8]
    xa = jax.nn.relu(o1a_ref[...] * jnp.dot(r8, ea,                                             preferred_element_type=jnp.float32))
    xb = jax.nn.relu(o1b_ref[...] * jnp.dot(r8, eb,                                             preferred_element_type=jnp.float32))
    xp = (jnp.dot(xa, wa_ref[...],                   preferred_element_type=jnp.float32)
          + jnp.dot(xb, wb_ref[...],                     preferred_element_type=jnp.float32))
    xp_ref[...] = xp
    als_ref[...] = jnp.dot(xp, as_ref[...],                            preferred_element_type=jnp.float32)
    ald_ref[...] = jnp.dot(xp, ad_ref[...],                            preferred_element_type=jnp.float32)


def _prep2(o1a, o1b, r1, W2pa, W2pb, As, Ad):
    BN = 1000
    return pl.pallas_call(
        _prep2_body,
        grid=(N // BN,),
        in_specs=[
            pl.BlockSpec((BN, 128), lambda i: (i, 0)),
            pl.BlockSpec((BN, 128), lambda i: (i, 0)),
            pl.BlockSpec((BN, 128), lambda i: (i, 0)),
            pl.BlockSpec((128, 128), lambda i: (0, 0)),
            pl.BlockSpec((128, 128), lambda i: (0, 0)),
            pl.BlockSpec((128, 128), lambda i: (0, 0)),
            pl.BlockSpec((128, 128), lambda i: (0, 0)),
        ],
        out_specs=[
            pl.BlockSpec((BN, 128), lambda i: (i, 0)),
            pl.BlockSpec((BN, 128), lambda i: (i, 0)),
            pl.BlockSpec((BN, 128), lambda i: (i, 0)),
        ],
        out_shape=[
            jax.ShapeDtypeStruct((N, 128), jnp.float32),
            jax.ShapeDtypeStruct((N, 128), jnp.float32),
            jax.ShapeDtypeStruct((N, 128), jnp.float32),
        ],
    )(o1a, o1b, r1, W2pa, W2pb, As, Ad)


def _recip_body(s_ref, r_ref):
    r_ref[...] = 1.0 / (s_ref[0] + s_ref[1] + 1e-16)


def _recip(s):
    BR = 2504
    return pl.pallas_call(
        _recip_body,
        grid=(NP16 // BR,),
        in_specs=[pl.BlockSpec((2, BR, 128), lambda i: (0, i, 0))],
        out_specs=pl.BlockSpec((BR, 128), lambda i: (i, 0)),
        out_shape=jax.ShapeDtypeStruct((NP16, 128), jnp.float32),
    )(s)


def _softmax_body(x_ref, r_ref, o_ref):
    z = x_ref[...][:, :OUT] * r_ref[...][:, 0:1]
    m = jnp.max(z, axis=-1, keepdims=True)
    e = jnp.exp(z - m)
    o_ref[...] = e / jnp.sum(e, axis=-1, keepdims=True)


def _softmax16(x, r2):
    BN = 2000
    return pl.pallas_call(
        _softmax_body,
        grid=(N // BN,),
        in_specs=[
            pl.BlockSpec((BN, 128), lambda i: (i, 0)),
            pl.BlockSpec((BN, 128), lambda i: (i, 0)),
        ],
        out_specs=pl.BlockSpec((BN, OUT), lambda i: (i, 0)),
        out_shape=jax.ShapeDtypeStruct((N, OUT), jnp.float32),
    )(x, r2)


def kernel(x, edge_index, walks, deg, W_in, b_in, w_se, Wq, Wk, Wv, Wo, g1,
           be1, g2, be2, Wf1, bf1, Wf2, bf2, w_pool, W1, a_src1, a_dst1, W2,
           a_src2, a_dst2):
    src = edge_index[0]
    dst = edge_index[1]
    padE = E_PAD - E
    src2d = jnp.concatenate(
        [src, jnp.zeros((padE,), jnp.int32)]).reshape(NROW, 128)
    dst2d = jnp.concatenate(
        [dst, jnp.full((padE,), N, jnp.int32)]).reshape(NROW, 128)
    widx2d = jnp.concatenate(
        [walks.reshape(-1), jnp.zeros((padE,), jnp.int32)]).reshape(NROW, 128)

    F = H1 * C1
    W_in_p = jnp.concatenate([W_in, jnp.zeros((IN, 128 - D))], axis=1)
    b_in_p = jnp.concatenate([b_in, jnp.zeros((128 - D,))]).reshape(1, 128)
    W2p = jnp.concatenate([W2, jnp.zeros((F, 128 - OUT))], axis=1)
    W2pa = W2p[:128]
    W2pb = W2p[128:]
    As1 = jnp.zeros((F, 128), jnp.float32).at[
        jnp.arange(F), jnp.arange(F) // C1].set(a_src1.reshape(-1))
    Ad1 = jnp.zeros((F, 128), jnp.float32).at[
        jnp.arange(F), jnp.arange(F) // C1].set(a_dst1.reshape(-1))
    As2 = jnp.zeros((128, 128), jnp.float32).at[
        jnp.arange(OUT), 0].set(a_src2.reshape(-1))
    Ad2 = jnp.zeros((128, 128), jnp.float32).at[
        jnp.arange(OUT), 0].set(a_dst2.reshape(-1))
    zpad = jnp.zeros((16, 128), jnp.float32)

    xw = _project(x, W_in_p, b_in_p)
    tok_pad = _sc_gather(xw, widx2d)
    gt = _walk_transformer(tok_pad, deg, w_se, Wq, Wk, Wv, Wo, g1, be1, g2,
                           be2, Wf1, bf1, Wf2, bf2, w_pool)

    xp1a, xp1b, als1, ald1 = _prep1(gt, W1, As1, Ad1)
    ex1, s1 = _sc_edge_stats(src2d, dst2d, als1,
                             jnp.concatenate([ald1, zpad]))
    r1 = _recip(s1)
    out1a = _sc_agg1a(src2d, dst2d, ex1, xp1a)
    out1b = _sc_agg1b(src2d, dst2d, ex1, xp1b)

    xp2, als2, ald2 = _prep2(out1a, out1b, r1[:N], W2pa, W2pb, As2, Ad2)
    ex2, s2 = _sc_edge_stats(src2d, dst2d, als2,
                             jnp.concatenate([ald2, zpad]))
    r2 = _recip(s2)
    out2 = _sc_agg2(src2d, dst2d, ex2, xp2)
    return _softmax16(out2, r2[:N])
